# baseline (speedup 1.0000x reference)
import numpy as np

import jax
import jax.numpy as jnp
from jax import lax
from jax.experimental import pallas as pl
from jax.experimental.pallas import tpu as pltpu
from jax.experimental.pallas import tpu_sc as plsc

_N = 10000
_M = 2500
_NPAD = 10240
_ROWS = _NPAD // 128
_R2 = 0.2 * 0.2
_R2F = float(np.float32(_R2))
_R2BITS = int(np.float32(_R2).view(np.int32))
_SENT = int(np.int32(0x7F000000))
_NEG_INF = float("-inf")

_MP = 2560
_NW = 32
_RPW = _MP // _NW
_NCH = _NPAD // 16
_C = 128
_NCHK = _RPW * 64 // _C


def _fps_kernel(px_ref, py_ref, pz_ref, idx_ref, qx_ref, qy_ref, qz_ref,
                mind_ref):
    lin = (jax.lax.broadcasted_iota(jnp.int32, (_ROWS, 128), 0) * 128
           + jax.lax.broadcasted_iota(jnp.int32, (_ROWS, 128), 1))
    valid = lin < _N
    px = px_ref[...]
    py = py_ref[...]
    pz = pz_ref[...]

    q0x = px_ref[0, 0]
    q0y = py_ref[0, 0]
    q0z = pz_ref[0, 0]
    dx = px - q0x
    dy = py - q0y
    dz = pz - q0z
    d0 = (dx * dx + dy * dy) + dz * dz
    mind_ref[...] = jnp.where(valid, d0, -1.0)
    idx_ref[0] = 0
    qx_ref[0] = q0x
    qy_ref[0] = q0y
    qz_ref[0] = q0z

    lane = jax.lax.broadcasted_iota(jnp.int32, (1, 128), 1)

    def body(i, q):
        qx, qy, qz = q
        ddx = px - qx
        ddy = py - qy
        ddz = pz - qz
        d = (ddx * ddx + ddy * ddy) + ddz * ddz
        mind2 = jnp.minimum(mind_ref[...], d)
        mind_ref[...] = mind2
        mx = jnp.max(mind2)
        nxt = jnp.min(jnp.where(mind2 == mx, lin, _NPAD))
        r = nxt >> 7
        c = nxt & 127
        lm = lane == c
        nqx = jnp.sum(jnp.where(lm, px_ref[pl.ds(r, 1), :], 0.0))
        nqy = jnp.sum(jnp.where(lm, py_ref[pl.ds(r, 1), :], 0.0))
        nqz = jnp.sum(jnp.where(lm, pz_ref[pl.ds(r, 1), :], 0.0))
        idx_ref[i] = nxt
        qx_ref[i] = nqx
        qy_ref[i] = nqy
        qz_ref[i] = nqz
        return (nqx, nqy, nqz)

    jax.lax.fori_loop(1, _M, body, (q0x, q0y, q0z))


def _fps(pos):
    coords = jnp.pad(pos, ((0, _NPAD - _N), (0, 0)))
    px = coords[:, 0].reshape(_ROWS, 128)
    py = coords[:, 1].reshape(_ROWS, 128)
    pz = coords[:, 2].reshape(_ROWS, 128)
    out_shape = (
        jax.ShapeDtypeStruct((_M,), jnp.int32),
        jax.ShapeDtypeStruct((_M,), jnp.float32),
        jax.ShapeDtypeStruct((_M,), jnp.float32),
        jax.ShapeDtypeStruct((_M,), jnp.float32),
    )
    idx, qx, qy, qz = pl.pallas_call(
        _fps_kernel,
        out_shape=out_shape,
        out_specs=tuple(pl.BlockSpec(memory_space=pltpu.SMEM)
                        for _ in range(4)),
        scratch_shapes=[pltpu.VMEM((_ROWS, 128), jnp.float32)],
    )(px, py, pz)
    return idx, jnp.stack([qx, qy, qz], axis=1)


def _bq_body(pxh, pyh, pzh, pqh, nbrh, vmh, rxh, ryh, rzh,
             pxv, pyv, pzv, pqv, cb, ci, nb, vb, rx, ry, rz):
    wid = lax.axis_index("s") * 2 + lax.axis_index("c")
    base = wid * _RPW
    pltpu.sync_copy(pxh, pxv)
    pltpu.sync_copy(pyh, pyv)
    pltpu.sync_copy(pzh, pzv)
    pltpu.sync_copy(pqh.at[pl.ds(base * 8, _RPW * 8)], pqv)

    i16 = lax.broadcasted_iota(jnp.int32, (16,), 0)
    z16 = jnp.zeros((16,), jnp.int32)
    ones16 = jnp.ones((16,), jnp.int32)
    zf16 = jnp.zeros((16,), jnp.float32)
    ninf16 = jnp.full((16,), _NEG_INF, jnp.float32)

    def row_body(t, _):
        qoff = z16 + t * 8
        qx = plsc.load_gather(pqv, [qoff])
        qy = plsc.load_gather(pqv, [qoff + 1])
        qz = plsc.load_gather(pqv, [qoff + 2])

        def one_chunk(c, cnt_v):
            sl = pl.ds(c * 16, 16)
            dx = pxv[sl] - qx
            dy = pyv[sl] - qy
            dz = pzv[sl] - qz
            d = (dx * dx + dy * dy) + dz * dz
            m = d < _R2F
            tgt = cnt_v + (plsc.cumsum(jnp.where(m, ones16, z16)) - 1)
            plsc.store_scatter(cb, [tgt], plsc.bitcast(d, jnp.int32), mask=m)
            plsc.store_scatter(ci, [tgt], c * 16 + i16, mask=m)
            return cnt_v + plsc.all_reduce_population_count(m)

        def dist_body(c2, cnt_v):
            cnt_v = one_chunk(c2 * 2, cnt_v)
            return one_chunk(c2 * 2 + 1, cnt_v)

        cnt = lax.fori_loop(0, _NCH // 2, dist_body, z16)[0]
        cb[pl.ds(cnt, 16)] = z16 + _SENT
        nv = (cnt + 15) >> 4

        def bs_body(k, lohi):
            lo, hi = lohi
            mid = (lo + hi) >> 1

            def cnt_body(j, acc):
                b = cb[pl.ds(j * 16, 16)]
                return acc + jnp.where(b <= mid, ones16, z16)

            cle = jnp.sum(lax.fori_loop(0, nv, cnt_body, z16))
            pred = cle >= 64
            return (jnp.where(pred, lo, mid + 1), jnp.where(pred, mid, hi))

        _, thr = lax.fori_loop(0, 30, bs_body,
                               (jnp.int32(0), jnp.int32(_R2BITS)))

        def lt_body(j, acc):
            b = cb[pl.ds(j * 16, 16)]
            return acc + jnp.where(b < thr, ones16, z16)

        cntlt = jnp.sum(lax.fori_loop(0, nv, lt_body, z16))
        quota = 64 - cntlt

        def emit_body(j, carry):
            outc, eqb = carry
            b = cb[pl.ds(j * 16, 16)]
            ii = ci[pl.ds(j * 16, 16)]
            ltm = b < thr
            eqm = b == thr
            eqc = plsc.cumsum(jnp.where(eqm, ones16, z16))
            take = ltm | (eqm & ((eqb + eqc) <= quota))
            plsc.store_compressed(nb.at[pl.ds(t * 64 + outc, 16)], ii,
                                  mask=take)
            outc = outc + plsc.all_reduce_population_count(take)[0]
            eqb = eqb + plsc.all_reduce_population_count(eqm)[0]
            return outc, eqb

        nsel, _ = lax.fori_loop(0, nv, emit_body,
                                (jnp.int32(0), jnp.int32(0)))

        for k in range(4):
            sl = pl.ds(t * 64 + k * 16, 16)
            slot = z16 + k * 16 + i16
            ok = slot < nsel
            idxv = jnp.where(ok, nb[sl], z16)
            nb[sl] = idxv
            vb[sl] = jnp.where(ok, zf16, ninf16)
            rx[sl] = plsc.load_gather(pxv, [idxv]) - qx
            ry[sl] = plsc.load_gather(pyv, [idxv]) - qy
            rz[sl] = plsc.load_gather(pzv, [idxv]) - qz
        return 0

    lax.fori_loop(0, _RPW, row_body, 0)
    sl = pl.ds(base * 64, _RPW * 64)
    pltpu.sync_copy(nb, nbrh.at[sl])
    pltpu.sync_copy(vb, vmh.at[sl])
    pltpu.sync_copy(rx, rxh.at[sl])
    pltpu.sync_copy(ry, ryh.at[sl])
    pltpu.sync_copy(rz, rzh.at[sl])


def _ballquery(px, py, pz, pqflat):
    mesh = plsc.VectorSubcoreMesh(core_axis_name="c", subcore_axis_name="s")
    f = pl.kernel(
        _bq_body,
        compiler_params=pltpu.CompilerParams(needs_layout_passes=False),
        out_type=(
            jax.ShapeDtypeStruct((_MP * 64,), jnp.int32),
            jax.ShapeDtypeStruct((_MP * 64,), jnp.float32),
            jax.ShapeDtypeStruct((_MP * 64,), jnp.float32),
            jax.ShapeDtypeStruct((_MP * 64,), jnp.float32),
            jax.ShapeDtypeStruct((_MP * 64,), jnp.float32),
        ),
        mesh=mesh,
        scratch_types=[
            pltpu.VMEM((_NPAD,), jnp.float32),
            pltpu.VMEM((_NPAD,), jnp.float32),
            pltpu.VMEM((_NPAD,), jnp.float32),
            pltpu.VMEM((_RPW * 8,), jnp.float32),
            pltpu.VMEM((_NPAD + 16,), jnp.int32),
            pltpu.VMEM((_NPAD + 16,), jnp.int32),
            pltpu.VMEM((_RPW * 64,), jnp.int32),
            pltpu.VMEM((_RPW * 64,), jnp.float32),
            pltpu.VMEM((_RPW * 64,), jnp.float32),
            pltpu.VMEM((_RPW * 64,), jnp.float32),
            pltpu.VMEM((_RPW * 64,), jnp.float32),
        ],
    )
    return f(px, py, pz, pqflat)


def _gather_body(xh, nbh, idxh, bh, xgh, bouth,
                 nbv, dstv, b0, b1, b2, b3, b4, b5, idxv, bbuf, gsem, ssem):
    wid = lax.axis_index("s") * 2 + lax.axis_index("c")
    base = wid * _RPW
    i16 = lax.broadcasted_iota(jnp.int32, (16,), 0)
    pltpu.sync_copy(nbh.at[pl.ds(base * 64, _RPW * 64)], nbv)

    def dst_body(k, _):
        e = k * 16 + i16
        t = e & 63
        iloc = e >> 6
        dst = t * _MP + base + iloc
        r = k >> 3
        c = (k & 7) * 16
        dstv[r, pl.ds(c, 16)] = dst
        return 0

    lax.fori_loop(0, _RPW * 64 // 16, dst_body, 0)

    pltpu.sync_copy(idxh.at[pl.ds(base, _RPW)], idxv)
    pltpu.async_copy(bh.at[idxv], bbuf, gsem).wait()
    pltpu.sync_copy(bbuf, bouth.at[pl.ds(base, _RPW)])

    bufs = (b0, b1, b2, b3, b4, b5)
    nbuf = len(bufs)
    ahead = 3

    def start_g(c):
        return pltpu.async_copy(xh.at[nbv.at[pl.ds(c * _C, _C)]],
                                bufs[c % nbuf], gsem)

    def start_s(c):
        return pltpu.async_copy(bufs[c % nbuf], xgh.at[dstv.at[c]], ssem)

    scat = [None] * _NCHK
    gat = [None] * _NCHK
    for c in range(min(ahead, _NCHK)):
        gat[c] = start_g(c)
    for c in range(_NCHK):
        n = c + ahead
        if n < _NCHK:
            if n - nbuf >= 0:
                scat[n - nbuf].wait()
            gat[n] = start_g(n)
        gat[c].wait()
        scat[c] = start_s(c)
    for c in range(max(_NCHK - nbuf, 0), _NCHK):
        scat[c].wait()


def _gather(x, nbf, idxp, batch):
    mesh = plsc.VectorSubcoreMesh(core_axis_name="c", subcore_axis_name="s")
    f = pl.kernel(
        _gather_body,
        compiler_params=pltpu.CompilerParams(needs_layout_passes=False),
        out_type=(
            jax.ShapeDtypeStruct((64 * _MP, 128), jnp.float32),
            jax.ShapeDtypeStruct((_MP,), jnp.int32),
        ),
        mesh=mesh,
        scratch_types=[
            pltpu.VMEM((_RPW * 64,), jnp.int32),
            pltpu.VMEM((_NCHK, _C), jnp.int32),
            pltpu.VMEM((_C, 128), jnp.float32),
            pltpu.VMEM((_C, 128), jnp.float32),
            pltpu.VMEM((_C, 128), jnp.float32),
            pltpu.VMEM((_C, 128), jnp.float32),
            pltpu.VMEM((_C, 128), jnp.float32),
            pltpu.VMEM((_C, 128), jnp.float32),
            pltpu.VMEM((_RPW,), jnp.int32),
            pltpu.VMEM((_RPW,), jnp.int32),
            pltpu.SemaphoreType.DMA,
            pltpu.SemaphoreType.DMA,
        ],
    )
    return f(x, nbf, idxp, batch)


def _mlp_kernel(xg_ref, rx_ref, ry_ref, rz_ref, vm_ref, w1_ref, w2_ref,
                aux_ref, o_ref):
    w1 = w1_ref[...]
    w2 = w2_ref[...]
    aux = aux_ref[...]
    acc = jnp.full((128, 128), _NEG_INF, jnp.float32)
    for t in range(64):
        xt = xg_ref[t]
        h = jnp.dot(xt, w1, preferred_element_type=jnp.float32)
        h = h + rx_ref[:, t:t + 1] * aux[0:1, :]
        h = h + ry_ref[:, t:t + 1] * aux[1:2, :]
        h = h + rz_ref[:, t:t + 1] * aux[2:3, :]
        h = jnp.maximum(h + aux[3:4, :], 0.0)
        h2 = jnp.dot(h, w2, preferred_element_type=jnp.float32)
        h2 = jnp.maximum(h2 + aux[4:5, :], 0.0)
        acc = jnp.maximum(acc, h2 + vm_ref[:, t:t + 1])
    o_ref[...] = acc


def _mlp(xg, rx2, ry2, rz2, vm2, w1a, w2, aux):
    grid = (_MP // 128,)
    return pl.pallas_call(
        _mlp_kernel,
        grid=grid,
        in_specs=[
            pl.BlockSpec((64, 128, 128), lambda i: (0, i, 0)),
            pl.BlockSpec((128, 64), lambda i: (i, 0)),
            pl.BlockSpec((128, 64), lambda i: (i, 0)),
            pl.BlockSpec((128, 64), lambda i: (i, 0)),
            pl.BlockSpec((128, 64), lambda i: (i, 0)),
            pl.BlockSpec((128, 128), lambda i: (0, 0)),
            pl.BlockSpec((128, 128), lambda i: (0, 0)),
            pl.BlockSpec((8, 128), lambda i: (0, 0)),
        ],
        out_specs=pl.BlockSpec((128, 128), lambda i: (i, 0)),
        out_shape=jax.ShapeDtypeStruct((_MP, 128), jnp.float32),
    )(xg, rx2, ry2, rz2, vm2, w1a, w2, aux)


def kernel(x, pos, batch, W1, b1, W2, b2):
    idx, pos_q = _fps(pos)

    big = jnp.float32(1e9)
    coords = jnp.concatenate(
        [pos, jnp.full((_NPAD - _N, 3), big, jnp.float32)], axis=0)
    px = coords[:, 0]
    py = coords[:, 1]
    pz = coords[:, 2]

    pq8 = jnp.full((_MP, 8), jnp.float32(2e9), jnp.float32)
    pq8 = pq8.at[:_M, 0:3].set(pos_q)
    pqflat = pq8.reshape(-1)

    nbf, vmf, rxf, ryf, rzf = _ballquery(px, py, pz, pqflat)

    idxp = jnp.zeros((_MP,), jnp.int32).at[:_M].set(idx)
    xg, bout = _gather(x, nbf, idxp, batch)

    aux = jnp.zeros((8, 128), jnp.float32)
    aux = aux.at[0:3, :].set(W1[128:131, :])
    aux = aux.at[3, :].set(b1)
    aux = aux.at[4, :].set(b2)

    out = _mlp(xg.reshape(64, _MP, 128),
               rxf.reshape(_MP, 64), ryf.reshape(_MP, 64),
               rzf.reshape(_MP, 64), vmf.reshape(_MP, 64),
               W1[:128, :], W2, aux)

    return (out[:_M], pos_q, bout[:_M])


# speedup vs baseline: 1.1944x; 1.1944x over previous
import numpy as np

import jax
import jax.numpy as jnp
from jax import lax
from jax.experimental import pallas as pl
from jax.experimental.pallas import tpu as pltpu
from jax.experimental.pallas import tpu_sc as plsc

_N = 10000
_M = 2500
_NPAD = 10240
_ROWS = _NPAD // 128
_R2 = 0.2 * 0.2
_R2F = float(np.float32(_R2))
_R2BITS = int(np.float32(_R2).view(np.int32))
_SENT = int(np.int32(0x7F000000))
_NEG_INF = float("-inf")

_MP = 2560
_NW = 32
_RPW = _MP // _NW
_NCH = _NPAD // 16
_C = 128
_NCHK = _RPW * 64 // _C


def _fps_kernel(px_ref, py_ref, pz_ref, idx_ref, qx_ref, qy_ref, qz_ref,
                mind_ref):
    lin = (jax.lax.broadcasted_iota(jnp.int32, (_ROWS, 128), 0) * 128
           + jax.lax.broadcasted_iota(jnp.int32, (_ROWS, 128), 1))
    valid = lin < _N
    px = px_ref[...]
    py = py_ref[...]
    pz = pz_ref[...]

    q0x = px_ref[0:1, 0:1]
    q0y = py_ref[0:1, 0:1]
    q0z = pz_ref[0:1, 0:1]
    dx = px - q0x
    dy = py - q0y
    dz = pz - q0z
    d0 = (dx * dx + dy * dy) + dz * dz
    mind_ref[...] = jnp.where(valid, d0, -1.0)
    idx_ref[0] = 0
    qx_ref[0] = px_ref[0, 0]
    qy_ref[0] = py_ref[0, 0]
    qz_ref[0] = pz_ref[0, 0]

    def body(i, q):
        qxb, qyb, qzb = q
        ddx = px - qxb
        ddy = py - qyb
        ddz = pz - qzb
        d = (ddx * ddx + ddy * ddy) + ddz * ddz
        mind2 = jnp.minimum(mind_ref[...], d)
        mind_ref[...] = mind2
        mx = jnp.max(jnp.max(mind2, axis=0, keepdims=True),
                     axis=1, keepdims=True)
        cand = jnp.where(mind2 == mx, lin, _NPAD)
        nxtv = jnp.min(jnp.min(cand, axis=0, keepdims=True),
                       axis=1, keepdims=True)
        sel = lin == nxtv
        nqx = jnp.sum(jnp.sum(jnp.where(sel, px, 0.0), axis=0,
                              keepdims=True), axis=1, keepdims=True)
        nqy = jnp.sum(jnp.sum(jnp.where(sel, py, 0.0), axis=0,
                              keepdims=True), axis=1, keepdims=True)
        nqz = jnp.sum(jnp.sum(jnp.where(sel, pz, 0.0), axis=0,
                              keepdims=True), axis=1, keepdims=True)
        idx_ref[i] = nxtv[0, 0]
        qx_ref[i] = nqx[0, 0]
        qy_ref[i] = nqy[0, 0]
        qz_ref[i] = nqz[0, 0]
        return (nqx, nqy, nqz)

    jax.lax.fori_loop(1, _M, body, (q0x, q0y, q0z))


def _fps(pos):
    coords = jnp.pad(pos, ((0, _NPAD - _N), (0, 0)))
    px = coords[:, 0].reshape(_ROWS, 128)
    py = coords[:, 1].reshape(_ROWS, 128)
    pz = coords[:, 2].reshape(_ROWS, 128)
    out_shape = (
        jax.ShapeDtypeStruct((_M,), jnp.int32),
        jax.ShapeDtypeStruct((_M,), jnp.float32),
        jax.ShapeDtypeStruct((_M,), jnp.float32),
        jax.ShapeDtypeStruct((_M,), jnp.float32),
    )
    idx, qx, qy, qz = pl.pallas_call(
        _fps_kernel,
        out_shape=out_shape,
        out_specs=tuple(pl.BlockSpec(memory_space=pltpu.SMEM)
                        for _ in range(4)),
        scratch_shapes=[pltpu.VMEM((_ROWS, 128), jnp.float32)],
    )(px, py, pz)
    return idx, jnp.stack([qx, qy, qz], axis=1)


def _bq_body(pxh, pyh, pzh, pqh, nbrh, vmh, rxh, ryh, rzh,
             pxv, pyv, pzv, pqv, cb, ci, nb, vb, rx, ry, rz):
    wid = lax.axis_index("s") * 2 + lax.axis_index("c")
    base = wid * _RPW
    pltpu.sync_copy(pxh, pxv)
    pltpu.sync_copy(pyh, pyv)
    pltpu.sync_copy(pzh, pzv)
    pltpu.sync_copy(pqh.at[pl.ds(base * 8, _RPW * 8)], pqv)

    i16 = lax.broadcasted_iota(jnp.int32, (16,), 0)
    z16 = jnp.zeros((16,), jnp.int32)
    ones16 = jnp.ones((16,), jnp.int32)
    zf16 = jnp.zeros((16,), jnp.float32)
    ninf16 = jnp.full((16,), _NEG_INF, jnp.float32)

    def row_body(t, _):
        qoff = z16 + t * 8
        qx = plsc.load_gather(pqv, [qoff])
        qy = plsc.load_gather(pqv, [qoff + 1])
        qz = plsc.load_gather(pqv, [qoff + 2])

        def one_chunk(c, cnt):
            sl = pl.ds(c * 16, 16)
            dx = pxv[sl] - qx
            dy = pyv[sl] - qy
            dz = pzv[sl] - qz
            d = (dx * dx + dy * dy) + dz * dz
            m = d < _R2F
            plsc.store_compressed(cb.at[pl.ds(cnt, 16)],
                                  plsc.bitcast(d, jnp.int32), mask=m)
            plsc.store_compressed(ci.at[pl.ds(cnt, 16)], c * 16 + i16,
                                  mask=m)
            return cnt + plsc.all_reduce_population_count(m)[0]

        def dist_body(c2, cnt):
            cnt = one_chunk(c2 * 2, cnt)
            return one_chunk(c2 * 2 + 1, cnt)

        cnt = lax.fori_loop(0, _NCH // 2, dist_body, jnp.int32(0))
        for k in range(4):
            cb[pl.ds(cnt + k * 16, 16)] = z16 + _SENT
        nv = (cnt + 15) >> 4
        nv4 = (cnt + 63) >> 6

        def bs_body(k, lohi):
            lo, hi = lohi
            mid = (lo + hi) >> 1

            def cnt_body(j, acc):
                for u in range(4):
                    b = cb[pl.ds(j * 64 + u * 16, 16)]
                    acc = acc + jnp.where(b <= mid, ones16, z16)
                return acc

            cle = jnp.sum(lax.fori_loop(0, nv4, cnt_body, z16))
            pred = cle >= 64
            return (jnp.where(pred, lo, mid + 1), jnp.where(pred, mid, hi))

        _, thr = lax.fori_loop(0, 30, bs_body,
                               (jnp.int32(0), jnp.int32(_R2BITS)))

        def lt_body(j, acc):
            b = cb[pl.ds(j * 16, 16)]
            return acc + jnp.where(b < thr, ones16, z16)

        cntlt = jnp.sum(lax.fori_loop(0, nv, lt_body, z16))
        quota = 64 - cntlt

        def emit_body(j, carry):
            outc, eqb = carry
            b = cb[pl.ds(j * 16, 16)]
            ii = ci[pl.ds(j * 16, 16)]
            ltm = b < thr
            eqm = b == thr
            eqc = plsc.cumsum(jnp.where(eqm, ones16, z16))
            take = ltm | (eqm & ((eqb + eqc) <= quota))
            plsc.store_compressed(nb.at[pl.ds(t * 64 + outc, 16)], ii,
                                  mask=take)
            outc = outc + plsc.all_reduce_population_count(take)[0]
            eqb = eqb + plsc.all_reduce_population_count(eqm)[0]
            return outc, eqb

        nsel, _ = lax.fori_loop(0, nv, emit_body,
                                (jnp.int32(0), jnp.int32(0)))

        for k in range(4):
            sl = pl.ds(t * 64 + k * 16, 16)
            slot = z16 + k * 16 + i16
            ok = slot < nsel
            idxv = jnp.where(ok, nb[sl], z16)
            nb[sl] = idxv
            vb[sl] = jnp.where(ok, zf16, ninf16)
            rx[sl] = plsc.load_gather(pxv, [idxv]) - qx
            ry[sl] = plsc.load_gather(pyv, [idxv]) - qy
            rz[sl] = plsc.load_gather(pzv, [idxv]) - qz
        return 0

    lax.fori_loop(0, _RPW, row_body, 0)
    sl = pl.ds(base * 64, _RPW * 64)
    pltpu.sync_copy(nb, nbrh.at[sl])
    pltpu.sync_copy(vb, vmh.at[sl])
    pltpu.sync_copy(rx, rxh.at[sl])
    pltpu.sync_copy(ry, ryh.at[sl])
    pltpu.sync_copy(rz, rzh.at[sl])


def _ballquery(px, py, pz, pqflat):
    mesh = plsc.VectorSubcoreMesh(core_axis_name="c", subcore_axis_name="s")
    f = pl.kernel(
        _bq_body,
        compiler_params=pltpu.CompilerParams(needs_layout_passes=False),
        out_type=(
            jax.ShapeDtypeStruct((_MP * 64,), jnp.int32),
            jax.ShapeDtypeStruct((_MP * 64,), jnp.float32),
            jax.ShapeDtypeStruct((_MP * 64,), jnp.float32),
            jax.ShapeDtypeStruct((_MP * 64,), jnp.float32),
            jax.ShapeDtypeStruct((_MP * 64,), jnp.float32),
        ),
        mesh=mesh,
        scratch_types=[
            pltpu.VMEM((_NPAD,), jnp.float32),
            pltpu.VMEM((_NPAD,), jnp.float32),
            pltpu.VMEM((_NPAD,), jnp.float32),
            pltpu.VMEM((_RPW * 8,), jnp.float32),
            pltpu.VMEM((_NPAD + 64,), jnp.int32),
            pltpu.VMEM((_NPAD + 16,), jnp.int32),
            pltpu.VMEM((_RPW * 64,), jnp.int32),
            pltpu.VMEM((_RPW * 64,), jnp.float32),
            pltpu.VMEM((_RPW * 64,), jnp.float32),
            pltpu.VMEM((_RPW * 64,), jnp.float32),
            pltpu.VMEM((_RPW * 64,), jnp.float32),
        ],
    )
    return f(px, py, pz, pqflat)


def _gather_body(xh, nbh, idxh, bh, xgh, bouth,
                 nbv, dstv, b0, b1, b2, b3, b4, b5, idxv, bbuf, gsem, ssem):
    wid = lax.axis_index("s") * 2 + lax.axis_index("c")
    base = wid * _RPW
    i16 = lax.broadcasted_iota(jnp.int32, (16,), 0)
    pltpu.sync_copy(nbh.at[pl.ds(base * 64, _RPW * 64)], nbv)

    def dst_body(k, _):
        e = k * 16 + i16
        t = e & 63
        iloc = e >> 6
        dst = t * _MP + base + iloc
        r = k >> 3
        c = (k & 7) * 16
        dstv[r, pl.ds(c, 16)] = dst
        return 0

    lax.fori_loop(0, _RPW * 64 // 16, dst_body, 0)

    pltpu.sync_copy(idxh.at[pl.ds(base, _RPW)], idxv)
    pltpu.async_copy(bh.at[idxv], bbuf, gsem).wait()
    pltpu.sync_copy(bbuf, bouth.at[pl.ds(base, _RPW)])

    bufs = (b0, b1, b2, b3, b4, b5)
    nbuf = len(bufs)
    ahead = 3

    def start_g(c):
        return pltpu.async_copy(xh.at[nbv.at[pl.ds(c * _C, _C)]],
                                bufs[c % nbuf], gsem)

    def start_s(c):
        return pltpu.async_copy(bufs[c % nbuf], xgh.at[dstv.at[c]], ssem)

    scat = [None] * _NCHK
    gat = [None] * _NCHK
    for c in range(min(ahead, _NCHK)):
        gat[c] = start_g(c)
    for c in range(_NCHK):
        n = c + ahead
        if n < _NCHK:
            if n - nbuf >= 0:
                scat[n - nbuf].wait()
            gat[n] = start_g(n)
        gat[c].wait()
        scat[c] = start_s(c)
    for c in range(max(_NCHK - nbuf, 0), _NCHK):
        scat[c].wait()


def _gather(x, nbf, idxp, batch):
    mesh = plsc.VectorSubcoreMesh(core_axis_name="c", subcore_axis_name="s")
    f = pl.kernel(
        _gather_body,
        compiler_params=pltpu.CompilerParams(needs_layout_passes=False),
        out_type=(
            jax.ShapeDtypeStruct((64 * _MP, 128), jnp.float32),
            jax.ShapeDtypeStruct((_MP,), jnp.int32),
        ),
        mesh=mesh,
        scratch_types=[
            pltpu.VMEM((_RPW * 64,), jnp.int32),
            pltpu.VMEM((_NCHK, _C), jnp.int32),
            pltpu.VMEM((_C, 128), jnp.float32),
            pltpu.VMEM((_C, 128), jnp.float32),
            pltpu.VMEM((_C, 128), jnp.float32),
            pltpu.VMEM((_C, 128), jnp.float32),
            pltpu.VMEM((_C, 128), jnp.float32),
            pltpu.VMEM((_C, 128), jnp.float32),
            pltpu.VMEM((_RPW,), jnp.int32),
            pltpu.VMEM((_RPW,), jnp.int32),
            pltpu.SemaphoreType.DMA,
            pltpu.SemaphoreType.DMA,
        ],
    )
    return f(x, nbf, idxp, batch)


def _mlp_kernel(xg_ref, rx_ref, ry_ref, rz_ref, vm_ref, w1_ref, w2_ref,
                aux_ref, o_ref):
    w1 = w1_ref[...]
    w2 = w2_ref[...]
    aux = aux_ref[...]
    acc = jnp.full((128, 128), _NEG_INF, jnp.float32)
    for t in range(64):
        xt = xg_ref[t]
        h = jnp.dot(xt, w1, preferred_element_type=jnp.float32)
        h = h + rx_ref[:, t:t + 1] * aux[0:1, :]
        h = h + ry_ref[:, t:t + 1] * aux[1:2, :]
        h = h + rz_ref[:, t:t + 1] * aux[2:3, :]
        h = jnp.maximum(h + aux[3:4, :], 0.0)
        h2 = jnp.dot(h, w2, preferred_element_type=jnp.float32)
        h2 = jnp.maximum(h2 + aux[4:5, :], 0.0)
        acc = jnp.maximum(acc, h2 + vm_ref[:, t:t + 1])
    o_ref[...] = acc


def _mlp(xg, rx2, ry2, rz2, vm2, w1a, w2, aux):
    grid = (_MP // 128,)
    return pl.pallas_call(
        _mlp_kernel,
        grid=grid,
        in_specs=[
            pl.BlockSpec((64, 128, 128), lambda i: (0, i, 0)),
            pl.BlockSpec((128, 64), lambda i: (i, 0)),
            pl.BlockSpec((128, 64), lambda i: (i, 0)),
            pl.BlockSpec((128, 64), lambda i: (i, 0)),
            pl.BlockSpec((128, 64), lambda i: (i, 0)),
            pl.BlockSpec((128, 128), lambda i: (0, 0)),
            pl.BlockSpec((128, 128), lambda i: (0, 0)),
            pl.BlockSpec((8, 128), lambda i: (0, 0)),
        ],
        out_specs=pl.BlockSpec((128, 128), lambda i: (i, 0)),
        out_shape=jax.ShapeDtypeStruct((_MP, 128), jnp.float32),
    )(xg, rx2, ry2, rz2, vm2, w1a, w2, aux)


def kernel(x, pos, batch, W1, b1, W2, b2):
    idx, pos_q = _fps(pos)

    big = jnp.float32(1e9)
    coords = jnp.concatenate(
        [pos, jnp.full((_NPAD - _N, 3), big, jnp.float32)], axis=0)
    px = coords[:, 0]
    py = coords[:, 1]
    pz = coords[:, 2]

    pq8 = jnp.full((_MP, 8), jnp.float32(2e9), jnp.float32)
    pq8 = pq8.at[:_M, 0:3].set(pos_q)
    pqflat = pq8.reshape(-1)

    nbf, vmf, rxf, ryf, rzf = _ballquery(px, py, pz, pqflat)

    idxp = jnp.zeros((_MP,), jnp.int32).at[:_M].set(idx)
    xg, bout = _gather(x, nbf, idxp, batch)

    aux = jnp.zeros((8, 128), jnp.float32)
    aux = aux.at[0:3, :].set(W1[128:131, :])
    aux = aux.at[3, :].set(b1)
    aux = aux.at[4, :].set(b2)

    out = _mlp(xg.reshape(64, _MP, 128),
               rxf.reshape(_MP, 64), ryf.reshape(_MP, 64),
               rzf.reshape(_MP, 64), vmf.reshape(_MP, 64),
               W1[:128, :], W2, aux)

    return (out[:_M], pos_q, bout[:_M])


# speedup vs baseline: 1.2219x; 1.0231x over previous
import numpy as np

import jax
import jax.numpy as jnp
from jax import lax
from jax.experimental import pallas as pl
from jax.experimental.pallas import tpu as pltpu
from jax.experimental.pallas import tpu_sc as plsc

_N = 10000
_M = 2500
_NPAD = 10240
_ROWS = _NPAD // 128
_R2 = 0.2 * 0.2
_R2F = float(np.float32(_R2))
_R2BITS = int(np.float32(_R2).view(np.int32))
_SENT = int(np.int32(0x7F000000))
_NEG_INF = float("-inf")

_MP = 2560
_NW = 32
_RPW = _MP // _NW
_NCH = _NPAD // 16
_C = 128
_NCHK = _RPW * 64 // _C


def _fps_kernel(px_ref, py_ref, pz_ref, idx_ref, qx_ref, qy_ref, qz_ref,
                mind_ref):
    lin = (jax.lax.broadcasted_iota(jnp.int32, (_ROWS, 128), 0) * 128
           + jax.lax.broadcasted_iota(jnp.int32, (_ROWS, 128), 1))
    valid = lin < _N
    px = px_ref[...]
    py = py_ref[...]
    pz = pz_ref[...]

    q0x = px_ref[0:1, 0:1]
    q0y = py_ref[0:1, 0:1]
    q0z = pz_ref[0:1, 0:1]
    dx = px - q0x
    dy = py - q0y
    dz = pz - q0z
    d0 = (dx * dx + dy * dy) + dz * dz
    mind_ref[...] = jnp.where(valid, d0, -1.0)
    idx_ref[0] = 0
    qx_ref[0] = px_ref[0, 0]
    qy_ref[0] = py_ref[0, 0]
    qz_ref[0] = pz_ref[0, 0]

    def body(i, q):
        qxb, qyb, qzb = q
        ddx = px - qxb
        ddy = py - qyb
        ddz = pz - qzb
        d = (ddx * ddx + ddy * ddy) + ddz * ddz
        mind2 = jnp.minimum(mind_ref[...], d)
        mind_ref[...] = mind2
        mx = jnp.max(jnp.max(mind2, axis=0, keepdims=True),
                     axis=1, keepdims=True)
        cand = jnp.where(mind2 == mx, lin, _NPAD)
        nxtv = jnp.min(jnp.min(cand, axis=0, keepdims=True),
                       axis=1, keepdims=True)
        nxt = nxtv[0, 0]
        r = nxt >> 7
        c = nxt & 127
        nqx = pltpu.roll(px_ref[pl.ds(r, 1), :], -c, 1)[0:1, 0:1]
        nqy = pltpu.roll(py_ref[pl.ds(r, 1), :], -c, 1)[0:1, 0:1]
        nqz = pltpu.roll(pz_ref[pl.ds(r, 1), :], -c, 1)[0:1, 0:1]
        idx_ref[i] = nxt
        qx_ref[i] = nqx[0, 0]
        qy_ref[i] = nqy[0, 0]
        qz_ref[i] = nqz[0, 0]
        return (nqx, nqy, nqz)

    jax.lax.fori_loop(1, _M, body, (q0x, q0y, q0z))


def _fps(pos):
    coords = jnp.pad(pos, ((0, _NPAD - _N), (0, 0)))
    px = coords[:, 0].reshape(_ROWS, 128)
    py = coords[:, 1].reshape(_ROWS, 128)
    pz = coords[:, 2].reshape(_ROWS, 128)
    out_shape = (
        jax.ShapeDtypeStruct((_M,), jnp.int32),
        jax.ShapeDtypeStruct((_M,), jnp.float32),
        jax.ShapeDtypeStruct((_M,), jnp.float32),
        jax.ShapeDtypeStruct((_M,), jnp.float32),
    )
    idx, qx, qy, qz = pl.pallas_call(
        _fps_kernel,
        out_shape=out_shape,
        out_specs=tuple(pl.BlockSpec(memory_space=pltpu.SMEM)
                        for _ in range(4)),
        scratch_shapes=[pltpu.VMEM((_ROWS, 128), jnp.float32)],
    )(px, py, pz)
    return idx, jnp.stack([qx, qy, qz], axis=1)


def _bq_body(pxh, pyh, pzh, pqh, nbrh, vmh, rxh, ryh, rzh,
             pxv, pyv, pzv, pqv, cb, ci, nb, vb, rx, ry, rz):
    wid = lax.axis_index("s") * 2 + lax.axis_index("c")
    base = wid * _RPW
    pltpu.sync_copy(pxh, pxv)
    pltpu.sync_copy(pyh, pyv)
    pltpu.sync_copy(pzh, pzv)
    pltpu.sync_copy(pqh.at[pl.ds(base * 8, _RPW * 8)], pqv)

    i16 = lax.broadcasted_iota(jnp.int32, (16,), 0)
    z16 = jnp.zeros((16,), jnp.int32)
    ones16 = jnp.ones((16,), jnp.int32)
    zf16 = jnp.zeros((16,), jnp.float32)
    ninf16 = jnp.full((16,), _NEG_INF, jnp.float32)

    def row_body(t, _):
        qoff = z16 + t * 8
        qx = plsc.load_gather(pqv, [qoff])
        qy = plsc.load_gather(pqv, [qoff + 1])
        qz = plsc.load_gather(pqv, [qoff + 2])

        def one_chunk(c, cnt):
            sl = pl.ds(c * 16, 16)
            dx = pxv[sl] - qx
            dy = pyv[sl] - qy
            dz = pzv[sl] - qz
            d = (dx * dx + dy * dy) + dz * dz
            m = d < _R2F
            plsc.store_compressed(cb.at[pl.ds(cnt, 16)],
                                  plsc.bitcast(d, jnp.int32), mask=m)
            plsc.store_compressed(ci.at[pl.ds(cnt, 16)], c * 16 + i16,
                                  mask=m)
            return cnt + plsc.all_reduce_population_count(m)[0]

        def dist_body(c4, cnt):
            cnt = one_chunk(c4 * 4, cnt)
            cnt = one_chunk(c4 * 4 + 1, cnt)
            cnt = one_chunk(c4 * 4 + 2, cnt)
            return one_chunk(c4 * 4 + 3, cnt)

        cnt = lax.fori_loop(0, _NCH // 4, dist_body, jnp.int32(0))
        for k in range(4):
            cb[pl.ds(cnt + k * 16, 16)] = z16 + _SENT
        nv = (cnt + 15) >> 4
        nv4 = (cnt + 63) >> 6

        def bs_body(k, lohi):
            lo, hi = lohi
            mid = (lo + hi) >> 1

            def cnt_body(j, acc):
                for u in range(4):
                    b = cb[pl.ds(j * 64 + u * 16, 16)]
                    acc = acc + jnp.where(b <= mid, ones16, z16)
                return acc

            cle = jnp.sum(lax.fori_loop(0, nv4, cnt_body, z16))
            pred = cle >= 64
            return (jnp.where(pred, lo, mid + 1), jnp.where(pred, mid, hi))

        _, thr = lax.fori_loop(0, 30, bs_body,
                               (jnp.int32(0), jnp.int32(_R2BITS)))

        def lt_body(j, acc):
            b = cb[pl.ds(j * 16, 16)]
            return acc + jnp.where(b < thr, ones16, z16)

        cntlt = jnp.sum(lax.fori_loop(0, nv, lt_body, z16))
        quota = 64 - cntlt

        def emit_body(j, carry):
            outc, eqb = carry
            b = cb[pl.ds(j * 16, 16)]
            ii = ci[pl.ds(j * 16, 16)]
            ltm = b < thr
            eqm = b == thr
            eqc = plsc.cumsum(jnp.where(eqm, ones16, z16))
            take = ltm | (eqm & ((eqb + eqc) <= quota))
            plsc.store_compressed(nb.at[pl.ds(t * 64 + outc, 16)], ii,
                                  mask=take)
            outc = outc + plsc.all_reduce_population_count(take)[0]
            eqb = eqb + plsc.all_reduce_population_count(eqm)[0]
            return outc, eqb

        nsel, _ = lax.fori_loop(0, nv, emit_body,
                                (jnp.int32(0), jnp.int32(0)))

        for k in range(4):
            sl = pl.ds(t * 64 + k * 16, 16)
            slot = z16 + k * 16 + i16
            ok = slot < nsel
            idxv = jnp.where(ok, nb[sl], z16)
            nb[sl] = idxv
            vb[sl] = jnp.where(ok, zf16, ninf16)
            rx[sl] = plsc.load_gather(pxv, [idxv]) - qx
            ry[sl] = plsc.load_gather(pyv, [idxv]) - qy
            rz[sl] = plsc.load_gather(pzv, [idxv]) - qz
        return 0

    lax.fori_loop(0, _RPW, row_body, 0)
    sl = pl.ds(base * 64, _RPW * 64)
    pltpu.sync_copy(nb, nbrh.at[sl])
    pltpu.sync_copy(vb, vmh.at[sl])
    pltpu.sync_copy(rx, rxh.at[sl])
    pltpu.sync_copy(ry, ryh.at[sl])
    pltpu.sync_copy(rz, rzh.at[sl])


def _ballquery(px, py, pz, pqflat):
    mesh = plsc.VectorSubcoreMesh(core_axis_name="c", subcore_axis_name="s")
    f = pl.kernel(
        _bq_body,
        compiler_params=pltpu.CompilerParams(needs_layout_passes=False),
        out_type=(
            jax.ShapeDtypeStruct((_MP * 64,), jnp.int32),
            jax.ShapeDtypeStruct((_MP * 64,), jnp.float32),
            jax.ShapeDtypeStruct((_MP * 64,), jnp.float32),
            jax.ShapeDtypeStruct((_MP * 64,), jnp.float32),
            jax.ShapeDtypeStruct((_MP * 64,), jnp.float32),
        ),
        mesh=mesh,
        scratch_types=[
            pltpu.VMEM((_NPAD,), jnp.float32),
            pltpu.VMEM((_NPAD,), jnp.float32),
            pltpu.VMEM((_NPAD,), jnp.float32),
            pltpu.VMEM((_RPW * 8,), jnp.float32),
            pltpu.VMEM((_NPAD + 64,), jnp.int32),
            pltpu.VMEM((_NPAD + 16,), jnp.int32),
            pltpu.VMEM((_RPW * 64,), jnp.int32),
            pltpu.VMEM((_RPW * 64,), jnp.float32),
            pltpu.VMEM((_RPW * 64,), jnp.float32),
            pltpu.VMEM((_RPW * 64,), jnp.float32),
            pltpu.VMEM((_RPW * 64,), jnp.float32),
        ],
    )
    return f(px, py, pz, pqflat)


_ICH = _MP // _C
_NU = _ICH * 2
_RING = 5


def _tr_kernel(a_ref, o_ref):
    o_ref[...] = a_ref[...].T


def _transpose_nb(nb2):
    return pl.pallas_call(
        _tr_kernel,
        grid=(_MP // 128,),
        in_specs=[pl.BlockSpec((128, 64), lambda i: (i, 0))],
        out_specs=pl.BlockSpec((64, 128), lambda i: (0, i)),
        out_shape=jax.ShapeDtypeStruct((64, _MP), jnp.int32),
    )(nb2)


def _gather_body(xh, nbth, idxh, bh, xgh, bouth,
                 idxm, b0, b1, b2, b3, b4, idxv, bbuf, gsem, ssem, isem):
    wid = lax.axis_index("s") * 2 + lax.axis_index("c")
    t0 = wid * 2
    base = wid * _RPW

    pltpu.sync_copy(idxh.at[pl.ds(base, _RPW)], idxv)
    pltpu.async_copy(bh.at[idxv], bbuf, gsem).wait()
    pltpu.sync_copy(bbuf, bouth.at[pl.ds(base, _RPW)])

    bufs = (b0, b1, b2, b3, b4)
    ilead = 4
    glead = 2

    def start_i(u):
        ic, t = u >> 1, u & 1
        return pltpu.async_copy(nbth.at[t0 + t, pl.ds(ic * _C, _C)],
                                idxm.at[u % _RING], isem)

    def start_g(u):
        return pltpu.async_copy(xh.at[idxm.at[u % _RING]],
                                bufs[u % _RING], gsem)

    def start_s(u):
        ic, t = u >> 1, u & 1
        row0 = (t0 + t) * _MP + ic * _C
        return pltpu.async_copy(bufs[u % _RING],
                                xgh.at[pl.ds(row0, _C)], ssem)

    idxd = [None] * _NU
    scat = [None] * _NU
    gat = [None] * _NU
    for m in range(ilead):
        idxd[m] = start_i(m)
    for m in range(glead):
        idxd[m].wait()
        gat[m] = start_g(m)
    for u in range(_NU):
        ni = u + ilead
        ng = u + glead
        if ni < _NU:
            idxd[ni] = start_i(ni)
        if ng < _NU:
            if ng - _RING >= 0:
                scat[ng - _RING].wait()
            idxd[ng].wait()
            gat[ng] = start_g(ng)
        gat[u].wait()
        scat[u] = start_s(u)
    for u in range(max(_NU - _RING, 0), _NU):
        scat[u].wait()


def _gather(x, nbt, idxp, batch):
    mesh = plsc.VectorSubcoreMesh(core_axis_name="c", subcore_axis_name="s")
    f = pl.kernel(
        _gather_body,
        compiler_params=pltpu.CompilerParams(needs_layout_passes=False),
        out_type=(
            jax.ShapeDtypeStruct((64 * _MP, 128), jnp.float32),
            jax.ShapeDtypeStruct((_MP,), jnp.int32),
        ),
        mesh=mesh,
        scratch_types=[
            pltpu.VMEM((_RING, _C), jnp.int32),
            pltpu.VMEM((_C, 128), jnp.float32),
            pltpu.VMEM((_C, 128), jnp.float32),
            pltpu.VMEM((_C, 128), jnp.float32),
            pltpu.VMEM((_C, 128), jnp.float32),
            pltpu.VMEM((_C, 128), jnp.float32),
            pltpu.VMEM((_RPW,), jnp.int32),
            pltpu.VMEM((_RPW,), jnp.int32),
            pltpu.SemaphoreType.DMA,
            pltpu.SemaphoreType.DMA,
            pltpu.SemaphoreType.DMA,
        ],
    )
    return f(x, nbt, idxp, batch)


def _mlp_kernel(xg_ref, rx_ref, ry_ref, rz_ref, vm_ref, w1_ref, w2_ref,
                aux_ref, o_ref):
    w1 = w1_ref[...]
    w2 = w2_ref[...]
    aux = aux_ref[...]
    acc = jnp.full((128, 128), _NEG_INF, jnp.float32)
    for t in range(64):
        xt = xg_ref[t]
        h = jnp.dot(xt, w1, preferred_element_type=jnp.float32)
        h = h + rx_ref[:, t:t + 1] * aux[0:1, :]
        h = h + ry_ref[:, t:t + 1] * aux[1:2, :]
        h = h + rz_ref[:, t:t + 1] * aux[2:3, :]
        h = jnp.maximum(h + aux[3:4, :], 0.0)
        h2 = jnp.dot(h, w2, preferred_element_type=jnp.float32)
        h2 = jnp.maximum(h2 + aux[4:5, :], 0.0)
        acc = jnp.maximum(acc, h2 + vm_ref[:, t:t + 1])
    o_ref[...] = acc


def _mlp(xg, rx2, ry2, rz2, vm2, w1a, w2, aux):
    grid = (_MP // 128,)
    return pl.pallas_call(
        _mlp_kernel,
        grid=grid,
        in_specs=[
            pl.BlockSpec((64, 128, 128), lambda i: (0, i, 0)),
            pl.BlockSpec((128, 64), lambda i: (i, 0)),
            pl.BlockSpec((128, 64), lambda i: (i, 0)),
            pl.BlockSpec((128, 64), lambda i: (i, 0)),
            pl.BlockSpec((128, 64), lambda i: (i, 0)),
            pl.BlockSpec((128, 128), lambda i: (0, 0)),
            pl.BlockSpec((128, 128), lambda i: (0, 0)),
            pl.BlockSpec((8, 128), lambda i: (0, 0)),
        ],
        out_specs=pl.BlockSpec((128, 128), lambda i: (i, 0)),
        out_shape=jax.ShapeDtypeStruct((_MP, 128), jnp.float32),
    )(xg, rx2, ry2, rz2, vm2, w1a, w2, aux)


def kernel(x, pos, batch, W1, b1, W2, b2):
    idx, pos_q = _fps(pos)

    big = jnp.float32(1e9)
    coords = jnp.concatenate(
        [pos, jnp.full((_NPAD - _N, 3), big, jnp.float32)], axis=0)
    px = coords[:, 0]
    py = coords[:, 1]
    pz = coords[:, 2]

    pq8 = jnp.full((_MP, 8), jnp.float32(2e9), jnp.float32)
    pq8 = pq8.at[:_M, 0:3].set(pos_q)
    pqflat = pq8.reshape(-1)

    nbf, vmf, rxf, ryf, rzf = _ballquery(px, py, pz, pqflat)

    idxp = jnp.zeros((_MP,), jnp.int32).at[:_M].set(idx)
    nbt = _transpose_nb(nbf.reshape(_MP, 64))
    xg, bout = _gather(x, nbt, idxp, batch)

    aux = jnp.zeros((8, 128), jnp.float32)
    aux = aux.at[0:3, :].set(W1[128:131, :])
    aux = aux.at[3, :].set(b1)
    aux = aux.at[4, :].set(b2)

    out = _mlp(xg.reshape(64, _MP, 128),
               rxf.reshape(_MP, 64), ryf.reshape(_MP, 64),
               rzf.reshape(_MP, 64), vmf.reshape(_MP, 64),
               W1[:128, :], W2, aux)

    return (out[:_M], pos_q, bout[:_M])


# speedup vs baseline: 1.3600x; 1.1130x over previous
import numpy as np

import jax
import jax.numpy as jnp
from jax import lax
from jax.experimental import pallas as pl
from jax.experimental.pallas import tpu as pltpu
from jax.experimental.pallas import tpu_sc as plsc

_N = 10000
_M = 2500
_NPAD = 10240
_ROWS = _NPAD // 128
_R2 = 0.2 * 0.2
_R2F = float(np.float32(_R2))
_R2BITS = int(np.float32(_R2).view(np.int32))
_SENT = int(np.int32(0x7F000000))
_NEG_INF = float("-inf")

_MP = 2560
_NW = 32
_RPW = _MP // _NW
_NCH = _NPAD // 16
_C = 128
_NCHK = _RPW * 64 // _C


_MR = _MP // 128


def _fps_kernel(px_ref, py_ref, pz_ref, idx_ref, qx_ref, qy_ref, qz_ref,
                mind_ref):
    lin = (jax.lax.broadcasted_iota(jnp.int32, (_ROWS, 128), 0) * 128
           + jax.lax.broadcasted_iota(jnp.int32, (_ROWS, 128), 1))
    lin20 = (jax.lax.broadcasted_iota(jnp.int32, (_MR, 128), 0) * 128
             + jax.lax.broadcasted_iota(jnp.int32, (_MR, 128), 1))
    valid = lin < _N
    px = px_ref[...]
    py = py_ref[...]
    pz = pz_ref[...]

    q0x = px_ref[0:1, 0:1]
    q0y = py_ref[0:1, 0:1]
    q0z = pz_ref[0:1, 0:1]
    dx = px - q0x
    dy = py - q0y
    dz = pz - q0z
    d0 = (dx * dx + dy * dy) + dz * dz
    mind_ref[...] = jnp.where(valid, d0, -1.0)
    m0 = lin20 == 0
    idx_ref[...] = jnp.zeros((_MR, 128), jnp.int32)
    qx_ref[...] = jnp.where(m0, jnp.broadcast_to(q0x, (_MR, 128)), 0.0)
    qy_ref[...] = jnp.where(m0, jnp.broadcast_to(q0y, (_MR, 128)), 0.0)
    qz_ref[...] = jnp.where(m0, jnp.broadcast_to(q0z, (_MR, 128)), 0.0)

    def body(i, q):
        qxb, qyb, qzb = q
        ddx = px - qxb
        ddy = py - qyb
        ddz = pz - qzb
        d = (ddx * ddx + ddy * ddy) + ddz * ddz
        mind2 = jnp.minimum(mind_ref[...], d)
        mind_ref[...] = mind2
        mx = jnp.max(jnp.max(mind2, axis=0, keepdims=True),
                     axis=1, keepdims=True)
        cand = jnp.where(mind2 == mx, lin, _NPAD)
        nxtv = jnp.min(jnp.min(cand, axis=0, keepdims=True),
                       axis=1, keepdims=True)
        nxt = nxtv[0, 0]
        r = nxt >> 7
        c = nxt & 127
        nqx = pltpu.roll(px_ref[pl.ds(r, 1), :], -c, 1)[0:1, 0:1]
        nqy = pltpu.roll(py_ref[pl.ds(r, 1), :], -c, 1)[0:1, 0:1]
        nqz = pltpu.roll(pz_ref[pl.ds(r, 1), :], -c, 1)[0:1, 0:1]
        mi = lin20 == i
        idx_ref[...] = jnp.where(mi, jnp.broadcast_to(nxtv, (_MR, 128)),
                                 idx_ref[...])
        qx_ref[...] = jnp.where(mi, jnp.broadcast_to(nqx, (_MR, 128)),
                                qx_ref[...])
        qy_ref[...] = jnp.where(mi, jnp.broadcast_to(nqy, (_MR, 128)),
                                qy_ref[...])
        qz_ref[...] = jnp.where(mi, jnp.broadcast_to(nqz, (_MR, 128)),
                                qz_ref[...])
        return (nqx, nqy, nqz)

    jax.lax.fori_loop(1, _M, body, (q0x, q0y, q0z))


def _fps(pos):
    coords = jnp.pad(pos, ((0, _NPAD - _N), (0, 0)))
    px = coords[:, 0].reshape(_ROWS, 128)
    py = coords[:, 1].reshape(_ROWS, 128)
    pz = coords[:, 2].reshape(_ROWS, 128)
    out_shape = (
        jax.ShapeDtypeStruct((_MR, 128), jnp.int32),
        jax.ShapeDtypeStruct((_MR, 128), jnp.float32),
        jax.ShapeDtypeStruct((_MR, 128), jnp.float32),
        jax.ShapeDtypeStruct((_MR, 128), jnp.float32),
    )
    idxb, qxb, qyb, qzb = pl.pallas_call(
        _fps_kernel,
        out_shape=out_shape,
        scratch_shapes=[pltpu.VMEM((_ROWS, 128), jnp.float32)],
    )(px, py, pz)
    idx = idxb.reshape(-1)[:_M]
    pos_q = jnp.stack([qxb.reshape(-1)[:_M], qyb.reshape(-1)[:_M],
                       qzb.reshape(-1)[:_M]], axis=1)
    return idx, pos_q


def _bq_body(pxh, pyh, pzh, pqh, nbrh, vmh, rxh, ryh, rzh,
             pxv, pyv, pzv, pqv, cb, ci, cb2, ci2, nb, vb, rx, ry, rz):
    wid = lax.axis_index("s") * 2 + lax.axis_index("c")
    base = wid * _RPW
    pltpu.sync_copy(pxh, pxv)
    pltpu.sync_copy(pyh, pyv)
    pltpu.sync_copy(pzh, pzv)
    pltpu.sync_copy(pqh.at[pl.ds(base * 8, _RPW * 8)], pqv)

    i16 = lax.broadcasted_iota(jnp.int32, (16,), 0)
    z16 = jnp.zeros((16,), jnp.int32)
    ones16 = jnp.ones((16,), jnp.int32)
    zf16 = jnp.zeros((16,), jnp.float32)
    ninf16 = jnp.full((16,), _NEG_INF, jnp.float32)

    def select_row(t, cbt, cit, cnt, qx, qy, qz):
        for k in range(4):
            cbt[pl.ds(cnt + k * 16, 16)] = z16 + _SENT
        nv = (cnt + 15) >> 4
        nv4 = (cnt + 63) >> 6

        def bs_body(k, lohi):
            lo, hi = lohi
            mid = (lo + hi) >> 1

            def cnt_body(j, acc):
                for u in range(4):
                    b = cbt[pl.ds(j * 64 + u * 16, 16)]
                    acc = acc + jnp.where(b <= mid, ones16, z16)
                return acc

            cle = jnp.sum(lax.fori_loop(0, nv4, cnt_body, z16))
            pred = cle >= 64
            return (jnp.where(pred, lo, mid + 1), jnp.where(pred, mid, hi))

        _, thr = lax.fori_loop(0, 30, bs_body,
                               (jnp.int32(0), jnp.int32(_R2BITS)))

        def lt_body(j, acc):
            b = cbt[pl.ds(j * 16, 16)]
            return acc + jnp.where(b < thr, ones16, z16)

        cntlt = jnp.sum(lax.fori_loop(0, nv, lt_body, z16))
        quota = 64 - cntlt

        def emit_body(j, carry):
            outc, eqb = carry
            b = cbt[pl.ds(j * 16, 16)]
            ii = cit[pl.ds(j * 16, 16)]
            ltm = b < thr
            eqm = b == thr
            eqc = plsc.cumsum(jnp.where(eqm, ones16, z16))
            take = ltm | (eqm & ((eqb + eqc) <= quota))
            plsc.store_compressed(nb.at[pl.ds(t * 64 + outc, 16)], ii,
                                  mask=take)
            outc = outc + plsc.all_reduce_population_count(take)[0]
            eqb = eqb + plsc.all_reduce_population_count(eqm)[0]
            return outc, eqb

        nsel, _ = lax.fori_loop(0, nv, emit_body,
                                (jnp.int32(0), jnp.int32(0)))

        for k in range(4):
            sl = pl.ds(t * 64 + k * 16, 16)
            slot = z16 + k * 16 + i16
            ok = slot < nsel
            idxv = jnp.where(ok, nb[sl], z16)
            nb[sl] = idxv
            vb[sl] = jnp.where(ok, zf16, ninf16)
            rx[sl] = plsc.load_gather(pxv, [idxv]) - qx
            ry[sl] = plsc.load_gather(pyv, [idxv]) - qy
            rz[sl] = plsc.load_gather(pzv, [idxv]) - qz

    def pair_body(tp, _):
        t = tp * 2
        qoff = z16 + t * 8
        qx0 = plsc.load_gather(pqv, [qoff])
        qy0 = plsc.load_gather(pqv, [qoff + 1])
        qz0 = plsc.load_gather(pqv, [qoff + 2])
        qx1 = plsc.load_gather(pqv, [qoff + 8])
        qy1 = plsc.load_gather(pqv, [qoff + 9])
        qz1 = plsc.load_gather(pqv, [qoff + 10])

        def one_chunk(c, carry):
            cnt0, cnt1 = carry
            sl = pl.ds(c * 16, 16)
            pxc = pxv[sl]
            pyc = pyv[sl]
            pzc = pzv[sl]
            lv = c * 16 + i16
            dx = pxc - qx0
            dy = pyc - qy0
            dz = pzc - qz0
            d0 = (dx * dx + dy * dy) + dz * dz
            m0 = d0 < _R2F
            plsc.store_compressed(cb.at[pl.ds(cnt0, 16)],
                                  plsc.bitcast(d0, jnp.int32), mask=m0)
            plsc.store_compressed(ci.at[pl.ds(cnt0, 16)], lv, mask=m0)
            ex = pxc - qx1
            ey = pyc - qy1
            ez = pzc - qz1
            d1 = (ex * ex + ey * ey) + ez * ez
            m1 = d1 < _R2F
            plsc.store_compressed(cb2.at[pl.ds(cnt1, 16)],
                                  plsc.bitcast(d1, jnp.int32), mask=m1)
            plsc.store_compressed(ci2.at[pl.ds(cnt1, 16)], lv, mask=m1)
            return (cnt0 + plsc.all_reduce_population_count(m0)[0],
                    cnt1 + plsc.all_reduce_population_count(m1)[0])

        def dist_body(c2, carry):
            carry = one_chunk(c2 * 2, carry)
            return one_chunk(c2 * 2 + 1, carry)

        cnt0, cnt1 = lax.fori_loop(0, _NCH // 2, dist_body,
                                   (jnp.int32(0), jnp.int32(0)))
        select_row(t, cb, ci, cnt0, qx0, qy0, qz0)
        select_row(t + 1, cb2, ci2, cnt1, qx1, qy1, qz1)
        return 0

    lax.fori_loop(0, _RPW // 2, pair_body, 0)
    sl = pl.ds(base * 64, _RPW * 64)
    pltpu.sync_copy(nb, nbrh.at[sl])
    pltpu.sync_copy(vb, vmh.at[sl])
    pltpu.sync_copy(rx, rxh.at[sl])
    pltpu.sync_copy(ry, ryh.at[sl])
    pltpu.sync_copy(rz, rzh.at[sl])


def _ballquery(px, py, pz, pqflat):
    mesh = plsc.VectorSubcoreMesh(core_axis_name="c", subcore_axis_name="s")
    f = pl.kernel(
        _bq_body,
        compiler_params=pltpu.CompilerParams(needs_layout_passes=False),
        out_type=(
            jax.ShapeDtypeStruct((_MP * 64,), jnp.int32),
            jax.ShapeDtypeStruct((_MP * 64,), jnp.float32),
            jax.ShapeDtypeStruct((_MP * 64,), jnp.float32),
            jax.ShapeDtypeStruct((_MP * 64,), jnp.float32),
            jax.ShapeDtypeStruct((_MP * 64,), jnp.float32),
        ),
        mesh=mesh,
        scratch_types=[
            pltpu.VMEM((_NPAD,), jnp.float32),
            pltpu.VMEM((_NPAD,), jnp.float32),
            pltpu.VMEM((_NPAD,), jnp.float32),
            pltpu.VMEM((_RPW * 8,), jnp.float32),
            pltpu.VMEM((_NPAD + 64,), jnp.int32),
            pltpu.VMEM((_NPAD + 16,), jnp.int32),
            pltpu.VMEM((_NPAD + 64,), jnp.int32),
            pltpu.VMEM((_NPAD + 16,), jnp.int32),
            pltpu.VMEM((_RPW * 64,), jnp.int32),
            pltpu.VMEM((_RPW * 64,), jnp.float32),
            pltpu.VMEM((_RPW * 64,), jnp.float32),
            pltpu.VMEM((_RPW * 64,), jnp.float32),
            pltpu.VMEM((_RPW * 64,), jnp.float32),
        ],
    )
    return f(px, py, pz, pqflat)


_ICH = _MP // _C
_NU = _ICH * 2
_RING = 5


def _tr_kernel(a_ref, o_ref):
    o_ref[...] = a_ref[...].T


def _transpose_nb(nb2):
    return pl.pallas_call(
        _tr_kernel,
        grid=(_MP // 128,),
        in_specs=[pl.BlockSpec((128, 64), lambda i: (i, 0))],
        out_specs=pl.BlockSpec((64, 128), lambda i: (0, i)),
        out_shape=jax.ShapeDtypeStruct((64, _MP), jnp.int32),
    )(nb2)


def _gather_body(xh, nbth, idxh, bh, xgh, bouth,
                 idxm, b0, b1, b2, b3, b4, idxv, bbuf, gsem, ssem, isem):
    wid = lax.axis_index("s") * 2 + lax.axis_index("c")
    t0 = wid * 2
    base = wid * _RPW

    pltpu.sync_copy(idxh.at[pl.ds(base, _RPW)], idxv)
    pltpu.async_copy(bh.at[idxv], bbuf, gsem).wait()
    pltpu.sync_copy(bbuf, bouth.at[pl.ds(base, _RPW)])

    bufs = (b0, b1, b2, b3, b4)
    ilead = 4
    glead = 2

    def start_i(u):
        ic, t = u >> 1, u & 1
        return pltpu.async_copy(nbth.at[t0 + t, pl.ds(ic * _C, _C)],
                                idxm.at[u % _RING], isem)

    def start_g(u):
        return pltpu.async_copy(xh.at[idxm.at[u % _RING]],
                                bufs[u % _RING], gsem)

    def start_s(u):
        ic, t = u >> 1, u & 1
        row0 = (t0 + t) * _MP + ic * _C
        return pltpu.async_copy(bufs[u % _RING],
                                xgh.at[pl.ds(row0, _C)], ssem)

    idxd = [None] * _NU
    scat = [None] * _NU
    gat = [None] * _NU
    for m in range(ilead):
        idxd[m] = start_i(m)
    for m in range(glead):
        idxd[m].wait()
        gat[m] = start_g(m)
    for u in range(_NU):
        ni = u + ilead
        ng = u + glead
        if ni < _NU:
            idxd[ni] = start_i(ni)
        if ng < _NU:
            if ng - _RING >= 0:
                scat[ng - _RING].wait()
            idxd[ng].wait()
            gat[ng] = start_g(ng)
        gat[u].wait()
        scat[u] = start_s(u)
    for u in range(max(_NU - _RING, 0), _NU):
        scat[u].wait()


def _gather(x, nbt, idxp, batch):
    mesh = plsc.VectorSubcoreMesh(core_axis_name="c", subcore_axis_name="s")
    f = pl.kernel(
        _gather_body,
        compiler_params=pltpu.CompilerParams(needs_layout_passes=False),
        out_type=(
            jax.ShapeDtypeStruct((64 * _MP, 128), jnp.float32),
            jax.ShapeDtypeStruct((_MP,), jnp.int32),
        ),
        mesh=mesh,
        scratch_types=[
            pltpu.VMEM((_RING, _C), jnp.int32),
            pltpu.VMEM((_C, 128), jnp.float32),
            pltpu.VMEM((_C, 128), jnp.float32),
            pltpu.VMEM((_C, 128), jnp.float32),
            pltpu.VMEM((_C, 128), jnp.float32),
            pltpu.VMEM((_C, 128), jnp.float32),
            pltpu.VMEM((_RPW,), jnp.int32),
            pltpu.VMEM((_RPW,), jnp.int32),
            pltpu.SemaphoreType.DMA,
            pltpu.SemaphoreType.DMA,
            pltpu.SemaphoreType.DMA,
        ],
    )
    return f(x, nbt, idxp, batch)


def _mlp_kernel(xg_ref, rx_ref, ry_ref, rz_ref, vm_ref, w1_ref, w2_ref,
                aux_ref, o_ref):
    w1 = w1_ref[...]
    w2 = w2_ref[...]
    aux = aux_ref[...]
    acc = jnp.full((128, 128), _NEG_INF, jnp.float32)
    for t in range(64):
        xt = xg_ref[t]
        h = jnp.dot(xt, w1, preferred_element_type=jnp.float32)
        h = h + rx_ref[:, t:t + 1] * aux[0:1, :]
        h = h + ry_ref[:, t:t + 1] * aux[1:2, :]
        h = h + rz_ref[:, t:t + 1] * aux[2:3, :]
        h = jnp.maximum(h + aux[3:4, :], 0.0)
        h2 = jnp.dot(h, w2, preferred_element_type=jnp.float32)
        h2 = jnp.maximum(h2 + aux[4:5, :], 0.0)
        acc = jnp.maximum(acc, h2 + vm_ref[:, t:t + 1])
    o_ref[...] = acc


def _mlp(xg, rx2, ry2, rz2, vm2, w1a, w2, aux):
    grid = (_MP // 128,)
    return pl.pallas_call(
        _mlp_kernel,
        grid=grid,
        in_specs=[
            pl.BlockSpec((64, 128, 128), lambda i: (0, i, 0)),
            pl.BlockSpec((128, 64), lambda i: (i, 0)),
            pl.BlockSpec((128, 64), lambda i: (i, 0)),
            pl.BlockSpec((128, 64), lambda i: (i, 0)),
            pl.BlockSpec((128, 64), lambda i: (i, 0)),
            pl.BlockSpec((128, 128), lambda i: (0, 0)),
            pl.BlockSpec((128, 128), lambda i: (0, 0)),
            pl.BlockSpec((8, 128), lambda i: (0, 0)),
        ],
        out_specs=pl.BlockSpec((128, 128), lambda i: (i, 0)),
        out_shape=jax.ShapeDtypeStruct((_MP, 128), jnp.float32),
    )(xg, rx2, ry2, rz2, vm2, w1a, w2, aux)


def kernel(x, pos, batch, W1, b1, W2, b2):
    idx, pos_q = _fps(pos)

    big = jnp.float32(1e9)
    coords = jnp.concatenate(
        [pos, jnp.full((_NPAD - _N, 3), big, jnp.float32)], axis=0)
    px = coords[:, 0]
    py = coords[:, 1]
    pz = coords[:, 2]

    pq8 = jnp.full((_MP, 8), jnp.float32(2e9), jnp.float32)
    pq8 = pq8.at[:_M, 0:3].set(pos_q)
    pqflat = pq8.reshape(-1)

    nbf, vmf, rxf, ryf, rzf = _ballquery(px, py, pz, pqflat)

    idxp = jnp.zeros((_MP,), jnp.int32).at[:_M].set(idx)
    nbt = _transpose_nb(nbf.reshape(_MP, 64))
    xg, bout = _gather(x, nbt, idxp, batch)

    aux = jnp.zeros((8, 128), jnp.float32)
    aux = aux.at[0:3, :].set(W1[128:131, :])
    aux = aux.at[3, :].set(b1)
    aux = aux.at[4, :].set(b2)

    out = _mlp(xg.reshape(64, _MP, 128),
               rxf.reshape(_MP, 64), ryf.reshape(_MP, 64),
               rzf.reshape(_MP, 64), vmf.reshape(_MP, 64),
               W1[:128, :], W2, aux)

    return (out[:_M], pos_q, bout[:_M])


# speedup vs baseline: 1.4201x; 1.0442x over previous
import numpy as np

import jax
import jax.numpy as jnp
from jax import lax
from jax.experimental import pallas as pl
from jax.experimental.pallas import tpu as pltpu
from jax.experimental.pallas import tpu_sc as plsc

_N = 10000
_M = 2500
_NPAD = 10240
_ROWS = _NPAD // 128
_R2 = 0.2 * 0.2
_R2F = float(np.float32(_R2))
_R2BITS = int(np.float32(_R2).view(np.int32))
_SENT = int(np.int32(0x7F000000))
_NEG_INF = float("-inf")

_MP = 2560
_NW = 32
_RPW = _MP // _NW
_NCH = _NPAD // 16
_C = 128
_NCHK = _RPW * 64 // _C


_MR = _MP // 128


def _fps_kernel(px_ref, py_ref, pz_ref, idx_ref, qx_ref, qy_ref, qz_ref,
                mind_ref):
    lin = (jax.lax.broadcasted_iota(jnp.int32, (_ROWS, 128), 0) * 128
           + jax.lax.broadcasted_iota(jnp.int32, (_ROWS, 128), 1))
    lin20 = (jax.lax.broadcasted_iota(jnp.int32, (_MR, 128), 0) * 128
             + jax.lax.broadcasted_iota(jnp.int32, (_MR, 128), 1))
    valid = lin < _N
    px = px_ref[...]
    py = py_ref[...]
    pz = pz_ref[...]

    q0x = px_ref[0:1, 0:1]
    q0y = py_ref[0:1, 0:1]
    q0z = pz_ref[0:1, 0:1]
    dx = px - q0x
    dy = py - q0y
    dz = pz - q0z
    d0 = (dx * dx + dy * dy) + dz * dz
    mind_ref[...] = jnp.where(valid, d0, -1.0)
    m0 = lin20 == 0
    idx_ref[...] = jnp.zeros((_MR, 128), jnp.int32)
    qx_ref[...] = jnp.where(m0, jnp.broadcast_to(q0x, (_MR, 128)), 0.0)
    qy_ref[...] = jnp.where(m0, jnp.broadcast_to(q0y, (_MR, 128)), 0.0)
    qz_ref[...] = jnp.where(m0, jnp.broadcast_to(q0z, (_MR, 128)), 0.0)

    def body(i, q):
        qxb, qyb, qzb = q
        ddx = px - qxb
        ddy = py - qyb
        ddz = pz - qzb
        d = (ddx * ddx + ddy * ddy) + ddz * ddz
        mind2 = jnp.minimum(mind_ref[...], d)
        mind_ref[...] = mind2
        mx = jnp.max(jnp.max(mind2, axis=0, keepdims=True),
                     axis=1, keepdims=True)
        cand = jnp.where(mind2 == mx, lin, _NPAD)
        nxtv = jnp.min(jnp.min(cand, axis=0, keepdims=True),
                       axis=1, keepdims=True)
        nxt = nxtv[0, 0]
        r = nxt >> 7
        c = nxt & 127
        nqx = pltpu.roll(px_ref[pl.ds(r, 1), :], -c, 1)[0:1, 0:1]
        nqy = pltpu.roll(py_ref[pl.ds(r, 1), :], -c, 1)[0:1, 0:1]
        nqz = pltpu.roll(pz_ref[pl.ds(r, 1), :], -c, 1)[0:1, 0:1]
        mi = lin20 == i
        idx_ref[...] = jnp.where(mi, jnp.broadcast_to(nxtv, (_MR, 128)),
                                 idx_ref[...])
        qx_ref[...] = jnp.where(mi, jnp.broadcast_to(nqx, (_MR, 128)),
                                qx_ref[...])
        qy_ref[...] = jnp.where(mi, jnp.broadcast_to(nqy, (_MR, 128)),
                                qy_ref[...])
        qz_ref[...] = jnp.where(mi, jnp.broadcast_to(nqz, (_MR, 128)),
                                qz_ref[...])
        return (nqx, nqy, nqz)

    jax.lax.fori_loop(1, _M, body, (q0x, q0y, q0z))


def _fps(pos):
    coords = jnp.pad(pos, ((0, _NPAD - _N), (0, 0)))
    px = coords[:, 0].reshape(_ROWS, 128)
    py = coords[:, 1].reshape(_ROWS, 128)
    pz = coords[:, 2].reshape(_ROWS, 128)
    out_shape = (
        jax.ShapeDtypeStruct((_MR, 128), jnp.int32),
        jax.ShapeDtypeStruct((_MR, 128), jnp.float32),
        jax.ShapeDtypeStruct((_MR, 128), jnp.float32),
        jax.ShapeDtypeStruct((_MR, 128), jnp.float32),
    )
    idxb, qxb, qyb, qzb = pl.pallas_call(
        _fps_kernel,
        out_shape=out_shape,
        scratch_shapes=[pltpu.VMEM((_ROWS, 128), jnp.float32)],
    )(px, py, pz)
    idxf = idxb.reshape(-1)
    qxf = qxb.reshape(-1)
    qyf = qyb.reshape(-1)
    qzf = qzb.reshape(-1)
    pos_q = jnp.stack([qxf[:_M], qyf[:_M], qzf[:_M]], axis=1)
    return idxf, qxf, qyf, qzf, pos_q


def _bq_body(pxh, pyh, pzh, qxh, qyh, qzh, nbrh, vmh, rxh, ryh, rzh,
             pxv, pyv, pzv, qxv, qyv, qzv, cb, ci, cb2, ci2,
             nb, vb, rx, ry, rz):
    wid = lax.axis_index("s") * 2 + lax.axis_index("c")
    base = wid * _RPW
    pltpu.sync_copy(pxh, pxv)
    pltpu.sync_copy(pyh, pyv)
    pltpu.sync_copy(pzh, pzv)
    sl = pl.ds(base, _RPW)
    pltpu.sync_copy(qxh.at[sl], qxv)
    pltpu.sync_copy(qyh.at[sl], qyv)
    pltpu.sync_copy(qzh.at[sl], qzv)

    i16 = lax.broadcasted_iota(jnp.int32, (16,), 0)
    z16 = jnp.zeros((16,), jnp.int32)
    ones16 = jnp.ones((16,), jnp.int32)
    zf16 = jnp.zeros((16,), jnp.float32)
    ninf16 = jnp.full((16,), _NEG_INF, jnp.float32)

    def select_row(t, cbt, cit, cnt, qx, qy, qz):
        for k in range(4):
            cbt[pl.ds(cnt + k * 16, 16)] = z16 + _SENT
        nv = (cnt + 15) >> 4
        nv4 = (cnt + 63) >> 6

        def bs_body(k, lohi):
            lo, hi = lohi
            mid = (lo + hi) >> 1

            def cnt_body(j, acc):
                for u in range(4):
                    b = cbt[pl.ds(j * 64 + u * 16, 16)]
                    acc = acc + jnp.where(b <= mid, ones16, z16)
                return acc

            cle = jnp.sum(lax.fori_loop(0, nv4, cnt_body, z16))
            pred = cle >= 64
            return (jnp.where(pred, lo, mid + 1), jnp.where(pred, mid, hi))

        _, thr = lax.fori_loop(0, 30, bs_body,
                               (jnp.int32(0), jnp.int32(_R2BITS)))

        def lt_body(j, acc):
            b = cbt[pl.ds(j * 16, 16)]
            return acc + jnp.where(b < thr, ones16, z16)

        cntlt = jnp.sum(lax.fori_loop(0, nv, lt_body, z16))
        quota = 64 - cntlt

        def emit_body(j, carry):
            outc, eqb = carry
            b = cbt[pl.ds(j * 16, 16)]
            ii = cit[pl.ds(j * 16, 16)]
            ltm = b < thr
            eqm = b == thr
            eqc = plsc.cumsum(jnp.where(eqm, ones16, z16))
            take = ltm | (eqm & ((eqb + eqc) <= quota))
            plsc.store_compressed(nb.at[pl.ds(t * 64 + outc, 16)], ii,
                                  mask=take)
            outc = outc + plsc.all_reduce_population_count(take)[0]
            eqb = eqb + plsc.all_reduce_population_count(eqm)[0]
            return outc, eqb

        nsel, _ = lax.fori_loop(0, nv, emit_body,
                                (jnp.int32(0), jnp.int32(0)))

        for k in range(4):
            sl = pl.ds(t * 64 + k * 16, 16)
            slot = z16 + k * 16 + i16
            ok = slot < nsel
            idxv = jnp.where(ok, nb[sl], z16)
            nb[sl] = idxv
            vb[sl] = jnp.where(ok, zf16, ninf16)
            rx[sl] = plsc.load_gather(pxv, [idxv]) - qx
            ry[sl] = plsc.load_gather(pyv, [idxv]) - qy
            rz[sl] = plsc.load_gather(pzv, [idxv]) - qz

    def pair_body(tp, _):
        t = tp * 2
        qoff = z16 + t
        qx0 = plsc.load_gather(qxv, [qoff])
        qy0 = plsc.load_gather(qyv, [qoff])
        qz0 = plsc.load_gather(qzv, [qoff])
        qx1 = plsc.load_gather(qxv, [qoff + 1])
        qy1 = plsc.load_gather(qyv, [qoff + 1])
        qz1 = plsc.load_gather(qzv, [qoff + 1])

        def one_chunk(c, carry):
            cnt0, cnt1 = carry
            sl = pl.ds(c * 16, 16)
            pxc = pxv[sl]
            pyc = pyv[sl]
            pzc = pzv[sl]
            lv = c * 16 + i16
            dx = pxc - qx0
            dy = pyc - qy0
            dz = pzc - qz0
            d0 = (dx * dx + dy * dy) + dz * dz
            m0 = d0 < _R2F
            plsc.store_compressed(cb.at[pl.ds(cnt0, 16)],
                                  plsc.bitcast(d0, jnp.int32), mask=m0)
            plsc.store_compressed(ci.at[pl.ds(cnt0, 16)], lv, mask=m0)
            ex = pxc - qx1
            ey = pyc - qy1
            ez = pzc - qz1
            d1 = (ex * ex + ey * ey) + ez * ez
            m1 = d1 < _R2F
            plsc.store_compressed(cb2.at[pl.ds(cnt1, 16)],
                                  plsc.bitcast(d1, jnp.int32), mask=m1)
            plsc.store_compressed(ci2.at[pl.ds(cnt1, 16)], lv, mask=m1)
            return (cnt0 + plsc.all_reduce_population_count(m0)[0],
                    cnt1 + plsc.all_reduce_population_count(m1)[0])

        def dist_body(c2, carry):
            carry = one_chunk(c2 * 2, carry)
            return one_chunk(c2 * 2 + 1, carry)

        cnt0, cnt1 = lax.fori_loop(0, _NCH // 2, dist_body,
                                   (jnp.int32(0), jnp.int32(0)))
        select_row(t, cb, ci, cnt0, qx0, qy0, qz0)
        select_row(t + 1, cb2, ci2, cnt1, qx1, qy1, qz1)
        return 0

    lax.fori_loop(0, _RPW // 2, pair_body, 0)
    sl = pl.ds(base * 64, _RPW * 64)
    pltpu.sync_copy(nb, nbrh.at[sl])
    pltpu.sync_copy(vb, vmh.at[sl])
    pltpu.sync_copy(rx, rxh.at[sl])
    pltpu.sync_copy(ry, ryh.at[sl])
    pltpu.sync_copy(rz, rzh.at[sl])


def _ballquery(px, py, pz, qxf, qyf, qzf):
    mesh = plsc.VectorSubcoreMesh(core_axis_name="c", subcore_axis_name="s")
    f = pl.kernel(
        _bq_body,
        compiler_params=pltpu.CompilerParams(needs_layout_passes=False),
        out_type=(
            jax.ShapeDtypeStruct((_MP * 64,), jnp.int32),
            jax.ShapeDtypeStruct((_MP * 64,), jnp.float32),
            jax.ShapeDtypeStruct((_MP * 64,), jnp.float32),
            jax.ShapeDtypeStruct((_MP * 64,), jnp.float32),
            jax.ShapeDtypeStruct((_MP * 64,), jnp.float32),
        ),
        mesh=mesh,
        scratch_types=[
            pltpu.VMEM((_NPAD,), jnp.float32),
            pltpu.VMEM((_NPAD,), jnp.float32),
            pltpu.VMEM((_NPAD,), jnp.float32),
            pltpu.VMEM((_RPW,), jnp.float32),
            pltpu.VMEM((_RPW,), jnp.float32),
            pltpu.VMEM((_RPW,), jnp.float32),
            pltpu.VMEM((_NPAD + 64,), jnp.int32),
            pltpu.VMEM((_NPAD + 16,), jnp.int32),
            pltpu.VMEM((_NPAD + 64,), jnp.int32),
            pltpu.VMEM((_NPAD + 16,), jnp.int32),
            pltpu.VMEM((_RPW * 64,), jnp.int32),
            pltpu.VMEM((_RPW * 64,), jnp.float32),
            pltpu.VMEM((_RPW * 64,), jnp.float32),
            pltpu.VMEM((_RPW * 64,), jnp.float32),
            pltpu.VMEM((_RPW * 64,), jnp.float32),
        ],
    )
    return f(px, py, pz, qxf, qyf, qzf)


_ICH = _MP // _C
_NU = _ICH * 2
_RING = 5


def _tr_kernel(a_ref, o_ref):
    o_ref[...] = a_ref[...].T


def _transpose_nb(nb2):
    return pl.pallas_call(
        _tr_kernel,
        grid=(_MP // 128,),
        in_specs=[pl.BlockSpec((128, 64), lambda i: (i, 0))],
        out_specs=pl.BlockSpec((64, 128), lambda i: (0, i)),
        out_shape=jax.ShapeDtypeStruct((64, _MP), jnp.int32),
    )(nb2)


def _gather_body(xh, nbth, idxh, bh, xgh, bouth,
                 idxm, b0, b1, b2, b3, b4, idxv, bbuf, gsem, ssem, isem):
    wid = lax.axis_index("s") * 2 + lax.axis_index("c")
    t0 = wid * 2
    base = wid * _RPW

    pltpu.sync_copy(idxh.at[pl.ds(base, _RPW)], idxv)
    pltpu.async_copy(bh.at[idxv], bbuf, gsem).wait()
    pltpu.sync_copy(bbuf, bouth.at[pl.ds(base, _RPW)])

    bufs = (b0, b1, b2, b3, b4)
    ilead = 4
    glead = 2

    def start_i(u):
        ic, t = u >> 1, u & 1
        return pltpu.async_copy(nbth.at[t0 + t, pl.ds(ic * _C, _C)],
                                idxm.at[u % _RING], isem)

    def start_g(u):
        return pltpu.async_copy(xh.at[idxm.at[u % _RING]],
                                bufs[u % _RING], gsem)

    def start_s(u):
        ic, t = u >> 1, u & 1
        row0 = (t0 + t) * _MP + ic * _C
        return pltpu.async_copy(bufs[u % _RING],
                                xgh.at[pl.ds(row0, _C)], ssem)

    idxd = [None] * _NU
    scat = [None] * _NU
    gat = [None] * _NU
    for m in range(ilead):
        idxd[m] = start_i(m)
    for m in range(glead):
        idxd[m].wait()
        gat[m] = start_g(m)
    for u in range(_NU):
        ni = u + ilead
        ng = u + glead
        if ni < _NU:
            idxd[ni] = start_i(ni)
        if ng < _NU:
            if ng - _RING >= 0:
                scat[ng - _RING].wait()
            idxd[ng].wait()
            gat[ng] = start_g(ng)
        gat[u].wait()
        scat[u] = start_s(u)
    for u in range(max(_NU - _RING, 0), _NU):
        scat[u].wait()


def _gather(x, nbt, idxp, batch):
    mesh = plsc.VectorSubcoreMesh(core_axis_name="c", subcore_axis_name="s")
    f = pl.kernel(
        _gather_body,
        compiler_params=pltpu.CompilerParams(needs_layout_passes=False),
        out_type=(
            jax.ShapeDtypeStruct((64 * _MP, 128), jnp.float32),
            jax.ShapeDtypeStruct((_MP,), jnp.int32),
        ),
        mesh=mesh,
        scratch_types=[
            pltpu.VMEM((_RING, _C), jnp.int32),
            pltpu.VMEM((_C, 128), jnp.float32),
            pltpu.VMEM((_C, 128), jnp.float32),
            pltpu.VMEM((_C, 128), jnp.float32),
            pltpu.VMEM((_C, 128), jnp.float32),
            pltpu.VMEM((_C, 128), jnp.float32),
            pltpu.VMEM((_RPW,), jnp.int32),
            pltpu.VMEM((_RPW,), jnp.int32),
            pltpu.SemaphoreType.DMA,
            pltpu.SemaphoreType.DMA,
            pltpu.SemaphoreType.DMA,
        ],
    )
    return f(x, nbt, idxp, batch)


def _mlp_kernel(xg_ref, rx_ref, ry_ref, rz_ref, vm_ref, w1_ref, w2_ref,
                aux_ref, o_ref):
    w1 = w1_ref[...]
    w2 = w2_ref[...]
    aux = aux_ref[...]
    acc = jnp.full((128, 128), _NEG_INF, jnp.float32)
    for t in range(64):
        xt = xg_ref[t]
        h = jnp.dot(xt, w1, preferred_element_type=jnp.float32)
        h = h + rx_ref[:, t:t + 1] * aux[0:1, :]
        h = h + ry_ref[:, t:t + 1] * aux[1:2, :]
        h = h + rz_ref[:, t:t + 1] * aux[2:3, :]
        h = jnp.maximum(h + aux[3:4, :], 0.0)
        h2 = jnp.dot(h, w2, preferred_element_type=jnp.float32)
        h2 = jnp.maximum(h2 + aux[4:5, :], 0.0)
        acc = jnp.maximum(acc, h2 + vm_ref[:, t:t + 1])
    o_ref[...] = acc


def _mlp(xg, rx2, ry2, rz2, vm2, w1a, w2, aux):
    grid = (_MP // 128,)
    return pl.pallas_call(
        _mlp_kernel,
        grid=grid,
        in_specs=[
            pl.BlockSpec((64, 128, 128), lambda i: (0, i, 0)),
            pl.BlockSpec((128, 64), lambda i: (i, 0)),
            pl.BlockSpec((128, 64), lambda i: (i, 0)),
            pl.BlockSpec((128, 64), lambda i: (i, 0)),
            pl.BlockSpec((128, 64), lambda i: (i, 0)),
            pl.BlockSpec((128, 128), lambda i: (0, 0)),
            pl.BlockSpec((128, 128), lambda i: (0, 0)),
            pl.BlockSpec((8, 128), lambda i: (0, 0)),
        ],
        out_specs=pl.BlockSpec((128, 128), lambda i: (i, 0)),
        out_shape=jax.ShapeDtypeStruct((_MP, 128), jnp.float32),
    )(xg, rx2, ry2, rz2, vm2, w1a, w2, aux)


def kernel(x, pos, batch, W1, b1, W2, b2):
    idxf, qxf, qyf, qzf, pos_q = _fps(pos)

    big = jnp.float32(1e9)
    coords = jnp.concatenate(
        [pos, jnp.full((_NPAD - _N, 3), big, jnp.float32)], axis=0)
    px = coords[:, 0]
    py = coords[:, 1]
    pz = coords[:, 2]

    nbf, vmf, rxf, ryf, rzf = _ballquery(px, py, pz, qxf, qyf, qzf)

    nbt = _transpose_nb(nbf.reshape(_MP, 64))
    xg, bout = _gather(x, nbt, idxf, batch)

    aux = jnp.zeros((8, 128), jnp.float32)
    aux = aux.at[0:3, :].set(W1[128:131, :])
    aux = aux.at[3, :].set(b1)
    aux = aux.at[4, :].set(b2)

    out = _mlp(xg.reshape(64, _MP, 128),
               rxf.reshape(_MP, 64), ryf.reshape(_MP, 64),
               rzf.reshape(_MP, 64), vmf.reshape(_MP, 64),
               W1[:128, :], W2, aux)

    return (out[:_M], pos_q, bout[:_M])


# speedup vs baseline: 1.5769x; 1.1104x over previous
import numpy as np

import jax
import jax.numpy as jnp
from jax import lax
from jax.experimental import pallas as pl
from jax.experimental.pallas import tpu as pltpu
from jax.experimental.pallas import tpu_sc as plsc

_N = 10000
_M = 2500
_NPAD = 10240
_ROWS = _NPAD // 128
_R2 = 0.2 * 0.2
_R2F = float(np.float32(_R2))
_R2BITS = int(np.float32(_R2).view(np.int32))
_SENT = int(np.int32(0x7F000000))
_NEG_INF = float("-inf")

_MP = 2560
_NW = 32
_RPW = _MP // _NW
_NCH = _NPAD // 16
_C = 128
_NCHK = _RPW * 64 // _C


_MR = _MP // 128


def _fps_kernel(px_ref, py_ref, pz_ref, idx_ref, qx_ref, qy_ref, qz_ref,
                mind_ref):
    lin = (jax.lax.broadcasted_iota(jnp.int32, (_ROWS, 128), 0) * 128
           + jax.lax.broadcasted_iota(jnp.int32, (_ROWS, 128), 1))
    lin20 = (jax.lax.broadcasted_iota(jnp.int32, (_MR, 128), 0) * 128
             + jax.lax.broadcasted_iota(jnp.int32, (_MR, 128), 1))
    linf = lin.astype(jnp.float32)
    valid = lin < _N
    px = px_ref[...]
    py = py_ref[...]
    pz = pz_ref[...]

    q0x = px_ref[0:1, 0:1]
    q0y = py_ref[0:1, 0:1]
    q0z = pz_ref[0:1, 0:1]
    dx = px - q0x
    dy = py - q0y
    dz = pz - q0z
    d0 = (dx * dx + dy * dy) + dz * dz
    mind_ref[...] = jnp.where(valid, d0, -1.0)
    m0 = lin20 == 0
    idx_ref[...] = jnp.zeros((_MR, 128), jnp.int32)
    qx_ref[...] = jnp.where(m0, jnp.broadcast_to(q0x, (_MR, 128)), 0.0)
    qy_ref[...] = jnp.where(m0, jnp.broadcast_to(q0y, (_MR, 128)), 0.0)
    qz_ref[...] = jnp.where(m0, jnp.broadcast_to(q0z, (_MR, 128)), 0.0)

    def body(i, q):
        qxb, qyb, qzb = q
        ddx = px - qxb
        ddy = py - qyb
        ddz = pz - qzb
        d = (ddx * ddx + ddy * ddy) + ddz * ddz
        mind2 = jnp.minimum(mind_ref[...], d)
        mind_ref[...] = mind2
        mx = jnp.max(jnp.max(mind2, axis=0, keepdims=True),
                     axis=1, keepdims=True)
        cand = jnp.where(mind2 == mx, linf, jnp.float32(_NPAD))
        nxtf = jnp.min(jnp.min(cand, axis=0, keepdims=True),
                       axis=1, keepdims=True)
        nxtv = nxtf.astype(jnp.int32)
        nxt = nxtv[0, 0]
        r = nxt >> 7
        c = nxt & 127
        nqx = pltpu.roll(px_ref[pl.ds(r, 1), :], -c, 1)[0:1, 0:1]
        nqy = pltpu.roll(py_ref[pl.ds(r, 1), :], -c, 1)[0:1, 0:1]
        nqz = pltpu.roll(pz_ref[pl.ds(r, 1), :], -c, 1)[0:1, 0:1]
        mi = lin20 == i
        idx_ref[...] = jnp.where(mi, jnp.broadcast_to(nxtv, (_MR, 128)),
                                 idx_ref[...])
        qx_ref[...] = jnp.where(mi, jnp.broadcast_to(nqx, (_MR, 128)),
                                qx_ref[...])
        qy_ref[...] = jnp.where(mi, jnp.broadcast_to(nqy, (_MR, 128)),
                                qy_ref[...])
        qz_ref[...] = jnp.where(mi, jnp.broadcast_to(nqz, (_MR, 128)),
                                qz_ref[...])
        return (nqx, nqy, nqz)

    jax.lax.fori_loop(1, _M, body, (q0x, q0y, q0z))


def _fps(pos):
    coords = jnp.pad(pos, ((0, _NPAD - _N), (0, 0)))
    px = coords[:, 0].reshape(_ROWS, 128)
    py = coords[:, 1].reshape(_ROWS, 128)
    pz = coords[:, 2].reshape(_ROWS, 128)
    out_shape = (
        jax.ShapeDtypeStruct((_MR, 128), jnp.int32),
        jax.ShapeDtypeStruct((_MR, 128), jnp.float32),
        jax.ShapeDtypeStruct((_MR, 128), jnp.float32),
        jax.ShapeDtypeStruct((_MR, 128), jnp.float32),
    )
    idxb, qxb, qyb, qzb = pl.pallas_call(
        _fps_kernel,
        out_shape=out_shape,
        scratch_shapes=[pltpu.VMEM((_ROWS, 128), jnp.float32)],
    )(px, py, pz)
    idxf = idxb.reshape(-1)
    qxf = qxb.reshape(-1)
    qyf = qyb.reshape(-1)
    qzf = qzb.reshape(-1)
    pos_q = jnp.stack([qxf[:_M], qyf[:_M], qzf[:_M]], axis=1)
    return idxf, qxf, qyf, qzf, pos_q


def _bq_body(pxh, pyh, pzh, qxh, qyh, qzh, nbrh, vmh, rxh, ryh, rzh,
             pxv, pyv, pzv, qxv, qyv, qzv, cb, ci, cb2, ci2,
             nb, vb, rx, ry, rz):
    wid = lax.axis_index("s") * 2 + lax.axis_index("c")
    base = wid * _RPW
    pltpu.sync_copy(pxh, pxv)
    pltpu.sync_copy(pyh, pyv)
    pltpu.sync_copy(pzh, pzv)
    sl = pl.ds(base, _RPW)
    pltpu.sync_copy(qxh.at[sl], qxv)
    pltpu.sync_copy(qyh.at[sl], qyv)
    pltpu.sync_copy(qzh.at[sl], qzv)

    i16 = lax.broadcasted_iota(jnp.int32, (16,), 0)
    z16 = jnp.zeros((16,), jnp.int32)
    ones16 = jnp.ones((16,), jnp.int32)
    zf16 = jnp.zeros((16,), jnp.float32)
    ninf16 = jnp.full((16,), _NEG_INF, jnp.float32)

    def select_row(t, cbt, cit, cnt, qx, qy, qz):
        for k in range(4):
            cbt[pl.ds(cnt + k * 16, 16)] = z16 + _SENT
        nv = (cnt + 15) >> 4
        nv4 = (cnt + 63) >> 6

        def bs_body(k, lohi):
            lo, hi = lohi
            mid = (lo + hi) >> 1

            def cnt_body(j, acc):
                for u in range(4):
                    b = cbt[pl.ds(j * 64 + u * 16, 16)]
                    acc = acc + jnp.where(b <= mid, ones16, z16)
                return acc

            cle = jnp.sum(lax.fori_loop(0, nv4, cnt_body, z16))
            pred = cle >= 64
            return (jnp.where(pred, lo, mid + 1), jnp.where(pred, mid, hi))

        _, thr = lax.fori_loop(0, 30, bs_body,
                               (jnp.int32(0), jnp.int32(_R2BITS)))

        def lt_body(j, acc):
            b = cbt[pl.ds(j * 16, 16)]
            return acc + jnp.where(b < thr, ones16, z16)

        cntlt = jnp.sum(lax.fori_loop(0, nv, lt_body, z16))
        quota = 64 - cntlt

        def emit_body(j, carry):
            outc, eqb = carry
            b = cbt[pl.ds(j * 16, 16)]
            ii = cit[pl.ds(j * 16, 16)]
            ltm = b < thr
            eqm = b == thr
            eqc = plsc.cumsum(jnp.where(eqm, ones16, z16))
            take = ltm | (eqm & ((eqb + eqc) <= quota))
            plsc.store_compressed(nb.at[pl.ds(t * 64 + outc, 16)], ii,
                                  mask=take)
            outc = outc + plsc.all_reduce_population_count(take)[0]
            eqb = eqb + plsc.all_reduce_population_count(eqm)[0]
            return outc, eqb

        nsel, _ = lax.fori_loop(0, nv, emit_body,
                                (jnp.int32(0), jnp.int32(0)))

        for k in range(4):
            sl = pl.ds(t * 64 + k * 16, 16)
            slot = z16 + k * 16 + i16
            ok = slot < nsel
            idxv = jnp.where(ok, nb[sl], z16)
            nb[sl] = idxv
            vb[sl] = jnp.where(ok, zf16, ninf16)
            rx[sl] = plsc.load_gather(pxv, [idxv]) - qx
            ry[sl] = plsc.load_gather(pyv, [idxv]) - qy
            rz[sl] = plsc.load_gather(pzv, [idxv]) - qz

    def pair_body(tp, _):
        t = tp * 2
        qoff = z16 + t
        qx0 = plsc.load_gather(qxv, [qoff])
        qy0 = plsc.load_gather(qyv, [qoff])
        qz0 = plsc.load_gather(qzv, [qoff])
        qx1 = plsc.load_gather(qxv, [qoff + 1])
        qy1 = plsc.load_gather(qyv, [qoff + 1])
        qz1 = plsc.load_gather(qzv, [qoff + 1])

        def one_chunk(c, carry):
            cnt0, cnt1 = carry
            sl = pl.ds(c * 16, 16)
            pxc = pxv[sl]
            pyc = pyv[sl]
            pzc = pzv[sl]
            lv = c * 16 + i16
            dx = pxc - qx0
            dy = pyc - qy0
            dz = pzc - qz0
            d0 = (dx * dx + dy * dy) + dz * dz
            m0 = d0 < _R2F
            plsc.store_compressed(cb.at[pl.ds(cnt0, 16)],
                                  plsc.bitcast(d0, jnp.int32), mask=m0)
            plsc.store_compressed(ci.at[pl.ds(cnt0, 16)], lv, mask=m0)
            ex = pxc - qx1
            ey = pyc - qy1
            ez = pzc - qz1
            d1 = (ex * ex + ey * ey) + ez * ez
            m1 = d1 < _R2F
            plsc.store_compressed(cb2.at[pl.ds(cnt1, 16)],
                                  plsc.bitcast(d1, jnp.int32), mask=m1)
            plsc.store_compressed(ci2.at[pl.ds(cnt1, 16)], lv, mask=m1)
            return (cnt0 + plsc.all_reduce_population_count(m0)[0],
                    cnt1 + plsc.all_reduce_population_count(m1)[0])

        def dist_body(c2, carry):
            carry = one_chunk(c2 * 2, carry)
            return one_chunk(c2 * 2 + 1, carry)

        cnt0, cnt1 = lax.fori_loop(0, _NCH // 2, dist_body,
                                   (jnp.int32(0), jnp.int32(0)))
        select_row(t, cb, ci, cnt0, qx0, qy0, qz0)
        select_row(t + 1, cb2, ci2, cnt1, qx1, qy1, qz1)
        return 0

    lax.fori_loop(0, _RPW // 2, pair_body, 0)
    sl = pl.ds(base * 64, _RPW * 64)
    pltpu.sync_copy(nb, nbrh.at[sl])
    pltpu.sync_copy(vb, vmh.at[sl])
    pltpu.sync_copy(rx, rxh.at[sl])
    pltpu.sync_copy(ry, ryh.at[sl])
    pltpu.sync_copy(rz, rzh.at[sl])


def _ballquery(px, py, pz, qxf, qyf, qzf):
    mesh = plsc.VectorSubcoreMesh(core_axis_name="c", subcore_axis_name="s")
    f = pl.kernel(
        _bq_body,
        compiler_params=pltpu.CompilerParams(needs_layout_passes=False),
        out_type=(
            jax.ShapeDtypeStruct((_MP * 64,), jnp.int32),
            jax.ShapeDtypeStruct((_MP * 64,), jnp.float32),
            jax.ShapeDtypeStruct((_MP * 64,), jnp.float32),
            jax.ShapeDtypeStruct((_MP * 64,), jnp.float32),
            jax.ShapeDtypeStruct((_MP * 64,), jnp.float32),
        ),
        mesh=mesh,
        scratch_types=[
            pltpu.VMEM((_NPAD,), jnp.float32),
            pltpu.VMEM((_NPAD,), jnp.float32),
            pltpu.VMEM((_NPAD,), jnp.float32),
            pltpu.VMEM((_RPW,), jnp.float32),
            pltpu.VMEM((_RPW,), jnp.float32),
            pltpu.VMEM((_RPW,), jnp.float32),
            pltpu.VMEM((_NPAD + 64,), jnp.int32),
            pltpu.VMEM((_NPAD + 16,), jnp.int32),
            pltpu.VMEM((_NPAD + 64,), jnp.int32),
            pltpu.VMEM((_NPAD + 16,), jnp.int32),
            pltpu.VMEM((_RPW * 64,), jnp.int32),
            pltpu.VMEM((_RPW * 64,), jnp.float32),
            pltpu.VMEM((_RPW * 64,), jnp.float32),
            pltpu.VMEM((_RPW * 64,), jnp.float32),
            pltpu.VMEM((_RPW * 64,), jnp.float32),
        ],
    )
    return f(px, py, pz, qxf, qyf, qzf)


_ICH = _MP // _C
_NU = _ICH * 2
_RING = 5


def _tr_kernel(a_ref, o_ref):
    o_ref[...] = a_ref[...].T


def _transpose_nb(nb2):
    return pl.pallas_call(
        _tr_kernel,
        grid=(_MP // 128,),
        in_specs=[pl.BlockSpec((128, 64), lambda i: (i, 0))],
        out_specs=pl.BlockSpec((64, 128), lambda i: (0, i)),
        out_shape=jax.ShapeDtypeStruct((64, _MP), jnp.int32),
    )(nb2)


def _gather_body(xh, nbth, idxh, bh, xgh, bouth,
                 idxm, b0, b1, b2, b3, b4, idxv, bbuf, gsem, ssem, isem):
    wid = lax.axis_index("s") * 2 + lax.axis_index("c")
    t0 = wid * 2
    base = wid * _RPW

    pltpu.sync_copy(idxh.at[pl.ds(base, _RPW)], idxv)
    pltpu.async_copy(bh.at[idxv], bbuf, gsem).wait()
    pltpu.sync_copy(bbuf, bouth.at[pl.ds(base, _RPW)])

    bufs = (b0, b1, b2, b3, b4)
    ilead = 4
    glead = 2

    def start_i(u):
        ic, t = u >> 1, u & 1
        return pltpu.async_copy(nbth.at[t0 + t, pl.ds(ic * _C, _C)],
                                idxm.at[u % _RING], isem)

    def start_g(u):
        return pltpu.async_copy(xh.at[idxm.at[u % _RING]],
                                bufs[u % _RING], gsem)

    def start_s(u):
        ic, t = u >> 1, u & 1
        row0 = (t0 + t) * _MP + ic * _C
        return pltpu.async_copy(bufs[u % _RING],
                                xgh.at[pl.ds(row0, _C)], ssem)

    idxd = [None] * _NU
    scat = [None] * _NU
    gat = [None] * _NU
    for m in range(ilead):
        idxd[m] = start_i(m)
    for m in range(glead):
        idxd[m].wait()
        gat[m] = start_g(m)
    for u in range(_NU):
        ni = u + ilead
        ng = u + glead
        if ni < _NU:
            idxd[ni] = start_i(ni)
        if ng < _NU:
            if ng - _RING >= 0:
                scat[ng - _RING].wait()
            idxd[ng].wait()
            gat[ng] = start_g(ng)
        gat[u].wait()
        scat[u] = start_s(u)
    for u in range(max(_NU - _RING, 0), _NU):
        scat[u].wait()


def _gather(x, nbt, idxp, batch):
    mesh = plsc.VectorSubcoreMesh(core_axis_name="c", subcore_axis_name="s")
    f = pl.kernel(
        _gather_body,
        compiler_params=pltpu.CompilerParams(needs_layout_passes=False),
        out_type=(
            jax.ShapeDtypeStruct((64 * _MP, 128), jnp.float32),
            jax.ShapeDtypeStruct((_MP,), jnp.int32),
        ),
        mesh=mesh,
        scratch_types=[
            pltpu.VMEM((_RING, _C), jnp.int32),
            pltpu.VMEM((_C, 128), jnp.float32),
            pltpu.VMEM((_C, 128), jnp.float32),
            pltpu.VMEM((_C, 128), jnp.float32),
            pltpu.VMEM((_C, 128), jnp.float32),
            pltpu.VMEM((_C, 128), jnp.float32),
            pltpu.VMEM((_RPW,), jnp.int32),
            pltpu.VMEM((_RPW,), jnp.int32),
            pltpu.SemaphoreType.DMA,
            pltpu.SemaphoreType.DMA,
            pltpu.SemaphoreType.DMA,
        ],
    )
    return f(x, nbt, idxp, batch)


def _mlp_kernel(xg_ref, rx_ref, ry_ref, rz_ref, vm_ref, w1_ref, w2_ref,
                aux_ref, o_ref):
    w1 = w1_ref[...]
    w2 = w2_ref[...]
    aux = aux_ref[...]
    acc = jnp.full((128, 128), _NEG_INF, jnp.float32)
    for t in range(64):
        xt = xg_ref[t]
        h = jnp.dot(xt, w1, preferred_element_type=jnp.float32)
        h = h + rx_ref[:, t:t + 1] * aux[0:1, :]
        h = h + ry_ref[:, t:t + 1] * aux[1:2, :]
        h = h + rz_ref[:, t:t + 1] * aux[2:3, :]
        h = jnp.maximum(h + aux[3:4, :], 0.0)
        h2 = jnp.dot(h, w2, preferred_element_type=jnp.float32)
        h2 = jnp.maximum(h2 + aux[4:5, :], 0.0)
        acc = jnp.maximum(acc, h2 + vm_ref[:, t:t + 1])
    o_ref[...] = acc


def _mlp(xg, rx2, ry2, rz2, vm2, w1a, w2, aux):
    grid = (_MP // 128,)
    return pl.pallas_call(
        _mlp_kernel,
        grid=grid,
        in_specs=[
            pl.BlockSpec((64, 128, 128), lambda i: (0, i, 0)),
            pl.BlockSpec((128, 64), lambda i: (i, 0)),
            pl.BlockSpec((128, 64), lambda i: (i, 0)),
            pl.BlockSpec((128, 64), lambda i: (i, 0)),
            pl.BlockSpec((128, 64), lambda i: (i, 0)),
            pl.BlockSpec((128, 128), lambda i: (0, 0)),
            pl.BlockSpec((128, 128), lambda i: (0, 0)),
            pl.BlockSpec((8, 128), lambda i: (0, 0)),
        ],
        out_specs=pl.BlockSpec((128, 128), lambda i: (i, 0)),
        out_shape=jax.ShapeDtypeStruct((_MP, 128), jnp.float32),
    )(xg, rx2, ry2, rz2, vm2, w1a, w2, aux)


def kernel(x, pos, batch, W1, b1, W2, b2):
    idxf, qxf, qyf, qzf, pos_q = _fps(pos)

    big = jnp.float32(1e9)
    coords = jnp.concatenate(
        [pos, jnp.full((_NPAD - _N, 3), big, jnp.float32)], axis=0)
    px = coords[:, 0]
    py = coords[:, 1]
    pz = coords[:, 2]

    nbf, vmf, rxf, ryf, rzf = _ballquery(px, py, pz, qxf, qyf, qzf)

    nbt = _transpose_nb(nbf.reshape(_MP, 64))
    xg, bout = _gather(x, nbt, idxf, batch)

    aux = jnp.zeros((8, 128), jnp.float32)
    aux = aux.at[0:3, :].set(W1[128:131, :])
    aux = aux.at[3, :].set(b1)
    aux = aux.at[4, :].set(b2)

    out = _mlp(xg.reshape(64, _MP, 128),
               rxf.reshape(_MP, 64), ryf.reshape(_MP, 64),
               rzf.reshape(_MP, 64), vmf.reshape(_MP, 64),
               W1[:128, :], W2, aux)

    return (out[:_M], pos_q, bout[:_M])


# speedup vs baseline: 1.6534x; 1.0485x over previous
import numpy as np

import jax
import jax.numpy as jnp
from jax import lax
from jax.experimental import pallas as pl
from jax.experimental.pallas import tpu as pltpu
from jax.experimental.pallas import tpu_sc as plsc

_N = 10000
_M = 2500
_NPAD = 10240
_ROWS = _NPAD // 128
_R2 = 0.2 * 0.2
_R2F = float(np.float32(_R2))
_R2BITS = int(np.float32(_R2).view(np.int32))
_SENT = int(np.int32(0x7F000000))
_NEG_INF = float("-inf")

_MP = 2560
_NW = 32
_RPW = _MP // _NW
_NCH = _NPAD // 16
_C = 128
_NCHK = _RPW * 64 // _C


_MR = _MP // 128


def _fps_kernel(px_ref, py_ref, pz_ref, idx_ref, qx_ref, qy_ref, qz_ref,
                mind_ref):
    lin = (jax.lax.broadcasted_iota(jnp.int32, (_ROWS, 128), 0) * 128
           + jax.lax.broadcasted_iota(jnp.int32, (_ROWS, 128), 1))
    lin20 = (jax.lax.broadcasted_iota(jnp.int32, (_MR, 128), 0) * 128
             + jax.lax.broadcasted_iota(jnp.int32, (_MR, 128), 1))
    linf = lin.astype(jnp.float32)
    valid = lin < _N
    px = px_ref[...]
    py = py_ref[...]
    pz = pz_ref[...]

    q0x = px_ref[0:1, 0:1]
    q0y = py_ref[0:1, 0:1]
    q0z = pz_ref[0:1, 0:1]
    dx = px - q0x
    dy = py - q0y
    dz = pz - q0z
    d0 = (dx * dx + dy * dy) + dz * dz
    mind_ref[...] = jnp.where(valid, d0, -1.0)
    m0 = lin20 == 0
    idx_ref[...] = jnp.zeros((_MR, 128), jnp.int32)
    qx_ref[...] = jnp.where(m0, jnp.broadcast_to(q0x, (_MR, 128)), 0.0)
    qy_ref[...] = jnp.where(m0, jnp.broadcast_to(q0y, (_MR, 128)), 0.0)
    qz_ref[...] = jnp.where(m0, jnp.broadcast_to(q0z, (_MR, 128)), 0.0)

    def body(i, q):
        qxb, qyb, qzb = q
        ddx = px - qxb
        ddy = py - qyb
        ddz = pz - qzb
        d = (ddx * ddx + ddy * ddy) + ddz * ddz
        mind2 = jnp.minimum(mind_ref[...], d)
        mind_ref[...] = mind2
        mx = jnp.max(jnp.max(mind2, axis=0, keepdims=True),
                     axis=1, keepdims=True)
        cand = jnp.where(mind2 == mx, linf, jnp.float32(_NPAD))
        nxtf = jnp.min(jnp.min(cand, axis=0, keepdims=True),
                       axis=1, keepdims=True)
        nxtv = nxtf.astype(jnp.int32)
        nxt = nxtv[0, 0]
        r = nxt >> 7
        c = nxt & 127
        nqx = pltpu.roll(px_ref[pl.ds(r, 1), :], -c, 1)[0:1, 0:1]
        nqy = pltpu.roll(py_ref[pl.ds(r, 1), :], -c, 1)[0:1, 0:1]
        nqz = pltpu.roll(pz_ref[pl.ds(r, 1), :], -c, 1)[0:1, 0:1]
        mi = lin20 == i
        idx_ref[...] = jnp.where(mi, jnp.broadcast_to(nxtv, (_MR, 128)),
                                 idx_ref[...])
        qx_ref[...] = jnp.where(mi, jnp.broadcast_to(nqx, (_MR, 128)),
                                qx_ref[...])
        qy_ref[...] = jnp.where(mi, jnp.broadcast_to(nqy, (_MR, 128)),
                                qy_ref[...])
        qz_ref[...] = jnp.where(mi, jnp.broadcast_to(nqz, (_MR, 128)),
                                qz_ref[...])
        return (nqx[0, 0], nqy[0, 0], nqz[0, 0])

    jax.lax.fori_loop(1, _M, body, (q0x[0, 0], q0y[0, 0], q0z[0, 0]))


def _fps(pos):
    coords = jnp.pad(pos, ((0, _NPAD - _N), (0, 0)))
    px = coords[:, 0].reshape(_ROWS, 128)
    py = coords[:, 1].reshape(_ROWS, 128)
    pz = coords[:, 2].reshape(_ROWS, 128)
    out_shape = (
        jax.ShapeDtypeStruct((_MR, 128), jnp.int32),
        jax.ShapeDtypeStruct((_MR, 128), jnp.float32),
        jax.ShapeDtypeStruct((_MR, 128), jnp.float32),
        jax.ShapeDtypeStruct((_MR, 128), jnp.float32),
    )
    idxb, qxb, qyb, qzb = pl.pallas_call(
        _fps_kernel,
        out_shape=out_shape,
        scratch_shapes=[pltpu.VMEM((_ROWS, 128), jnp.float32)],
    )(px, py, pz)
    idxf = idxb.reshape(-1)
    qxf = qxb.reshape(-1)
    qyf = qyb.reshape(-1)
    qzf = qzb.reshape(-1)
    pos_q = jnp.stack([qxf[:_M], qyf[:_M], qzf[:_M]], axis=1)
    return idxf, qxf, qyf, qzf, pos_q


def _bq_body(pxh, pyh, pzh, qxh, qyh, qzh, nbrh, vmh, rxh, ryh, rzh,
             pxv, pyv, pzv, qxv, qyv, qzv, cb, ci, cb2, ci2,
             nb, vb, rx, ry, rz):
    wid = lax.axis_index("s") * 2 + lax.axis_index("c")
    base = wid * _RPW
    pltpu.sync_copy(pxh, pxv)
    pltpu.sync_copy(pyh, pyv)
    pltpu.sync_copy(pzh, pzv)
    sl = pl.ds(base, _RPW)
    pltpu.sync_copy(qxh.at[sl], qxv)
    pltpu.sync_copy(qyh.at[sl], qyv)
    pltpu.sync_copy(qzh.at[sl], qzv)

    i16 = lax.broadcasted_iota(jnp.int32, (16,), 0)
    z16 = jnp.zeros((16,), jnp.int32)
    ones16 = jnp.ones((16,), jnp.int32)
    zf16 = jnp.zeros((16,), jnp.float32)
    ninf16 = jnp.full((16,), _NEG_INF, jnp.float32)

    def select_row(t, cbt, cit, cnt, qx, qy, qz):
        for k in range(4):
            cbt[pl.ds(cnt + k * 16, 16)] = z16 + _SENT
        nv = (cnt + 15) >> 4
        nv4 = (cnt + 63) >> 6

        def bs_body(k, lohi):
            lo, hi = lohi
            mid = (lo + hi) >> 1

            def cnt_body(j, acc):
                for u in range(4):
                    b = cbt[pl.ds(j * 64 + u * 16, 16)]
                    acc = acc + jnp.where(b <= mid, ones16, z16)
                return acc

            cle = jnp.sum(lax.fori_loop(0, nv4, cnt_body, z16))
            pred = cle >= 64
            return (jnp.where(pred, lo, mid + 1), jnp.where(pred, mid, hi))

        _, thr = lax.fori_loop(0, 30, bs_body,
                               (jnp.int32(0), jnp.int32(_R2BITS)))

        def lt_body(j, acc):
            b = cbt[pl.ds(j * 16, 16)]
            return acc + jnp.where(b < thr, ones16, z16)

        cntlt = jnp.sum(lax.fori_loop(0, nv, lt_body, z16))
        quota = 64 - cntlt

        def emit_body(j, carry):
            outc, eqb = carry
            b = cbt[pl.ds(j * 16, 16)]
            ii = cit[pl.ds(j * 16, 16)]
            ltm = b < thr
            eqm = b == thr
            eqc = plsc.cumsum(jnp.where(eqm, ones16, z16))
            take = ltm | (eqm & ((eqb + eqc) <= quota))
            plsc.store_compressed(nb.at[pl.ds(t * 64 + outc, 16)], ii,
                                  mask=take)
            outc = outc + plsc.all_reduce_population_count(take)[0]
            eqb = eqb + plsc.all_reduce_population_count(eqm)[0]
            return outc, eqb

        nsel, _ = lax.fori_loop(0, nv, emit_body,
                                (jnp.int32(0), jnp.int32(0)))

        for k in range(4):
            sl = pl.ds(t * 64 + k * 16, 16)
            slot = z16 + k * 16 + i16
            ok = slot < nsel
            idxv = jnp.where(ok, nb[sl], z16)
            nb[sl] = idxv
            vb[sl] = jnp.where(ok, zf16, ninf16)
            rx[sl] = plsc.load_gather(pxv, [idxv]) - qx
            ry[sl] = plsc.load_gather(pyv, [idxv]) - qy
            rz[sl] = plsc.load_gather(pzv, [idxv]) - qz

    def pair_body(tp, _):
        t = tp * 2
        qoff = z16 + t
        qx0 = plsc.load_gather(qxv, [qoff])
        qy0 = plsc.load_gather(qyv, [qoff])
        qz0 = plsc.load_gather(qzv, [qoff])
        qx1 = plsc.load_gather(qxv, [qoff + 1])
        qy1 = plsc.load_gather(qyv, [qoff + 1])
        qz1 = plsc.load_gather(qzv, [qoff + 1])

        def one_chunk(c, carry):
            cnt0, cnt1 = carry
            sl = pl.ds(c * 16, 16)
            pxc = pxv[sl]
            pyc = pyv[sl]
            pzc = pzv[sl]
            lv = c * 16 + i16
            dx = pxc - qx0
            dy = pyc - qy0
            dz = pzc - qz0
            d0 = (dx * dx + dy * dy) + dz * dz
            m0 = d0 < _R2F
            plsc.store_compressed(cb.at[pl.ds(cnt0, 16)],
                                  plsc.bitcast(d0, jnp.int32), mask=m0)
            plsc.store_compressed(ci.at[pl.ds(cnt0, 16)], lv, mask=m0)
            ex = pxc - qx1
            ey = pyc - qy1
            ez = pzc - qz1
            d1 = (ex * ex + ey * ey) + ez * ez
            m1 = d1 < _R2F
            plsc.store_compressed(cb2.at[pl.ds(cnt1, 16)],
                                  plsc.bitcast(d1, jnp.int32), mask=m1)
            plsc.store_compressed(ci2.at[pl.ds(cnt1, 16)], lv, mask=m1)
            return (cnt0 + plsc.all_reduce_population_count(m0)[0],
                    cnt1 + plsc.all_reduce_population_count(m1)[0])

        def dist_body(c2, carry):
            carry = one_chunk(c2 * 2, carry)
            return one_chunk(c2 * 2 + 1, carry)

        cnt0, cnt1 = lax.fori_loop(0, _NCH // 2, dist_body,
                                   (jnp.int32(0), jnp.int32(0)))
        select_row(t, cb, ci, cnt0, qx0, qy0, qz0)
        select_row(t + 1, cb2, ci2, cnt1, qx1, qy1, qz1)
        return 0

    lax.fori_loop(0, _RPW // 2, pair_body, 0)
    sl = pl.ds(base * 64, _RPW * 64)
    pltpu.sync_copy(nb, nbrh.at[sl])
    pltpu.sync_copy(vb, vmh.at[sl])
    pltpu.sync_copy(rx, rxh.at[sl])
    pltpu.sync_copy(ry, ryh.at[sl])
    pltpu.sync_copy(rz, rzh.at[sl])


def _ballquery(px, py, pz, qxf, qyf, qzf):
    mesh = plsc.VectorSubcoreMesh(core_axis_name="c", subcore_axis_name="s")
    f = pl.kernel(
        _bq_body,
        compiler_params=pltpu.CompilerParams(needs_layout_passes=False),
        out_type=(
            jax.ShapeDtypeStruct((_MP * 64,), jnp.int32),
            jax.ShapeDtypeStruct((_MP * 64,), jnp.float32),
            jax.ShapeDtypeStruct((_MP * 64,), jnp.float32),
            jax.ShapeDtypeStruct((_MP * 64,), jnp.float32),
            jax.ShapeDtypeStruct((_MP * 64,), jnp.float32),
        ),
        mesh=mesh,
        scratch_types=[
            pltpu.VMEM((_NPAD,), jnp.float32),
            pltpu.VMEM((_NPAD,), jnp.float32),
            pltpu.VMEM((_NPAD,), jnp.float32),
            pltpu.VMEM((_RPW,), jnp.float32),
            pltpu.VMEM((_RPW,), jnp.float32),
            pltpu.VMEM((_RPW,), jnp.float32),
            pltpu.VMEM((_NPAD + 64,), jnp.int32),
            pltpu.VMEM((_NPAD + 16,), jnp.int32),
            pltpu.VMEM((_NPAD + 64,), jnp.int32),
            pltpu.VMEM((_NPAD + 16,), jnp.int32),
            pltpu.VMEM((_RPW * 64,), jnp.int32),
            pltpu.VMEM((_RPW * 64,), jnp.float32),
            pltpu.VMEM((_RPW * 64,), jnp.float32),
            pltpu.VMEM((_RPW * 64,), jnp.float32),
            pltpu.VMEM((_RPW * 64,), jnp.float32),
        ],
    )
    return f(px, py, pz, qxf, qyf, qzf)


_ICH = _MP // _C
_NU = _ICH * 2
_RING = 5


def _tr_kernel(a_ref, o_ref):
    o_ref[...] = a_ref[...].T


def _transpose_nb(nb2):
    return pl.pallas_call(
        _tr_kernel,
        grid=(_MP // 128,),
        in_specs=[pl.BlockSpec((128, 64), lambda i: (i, 0))],
        out_specs=pl.BlockSpec((64, 128), lambda i: (0, i)),
        out_shape=jax.ShapeDtypeStruct((64, _MP), jnp.int32),
    )(nb2)


def _gather_body(xh, nbth, idxh, bh, xgh, bouth,
                 idxm, b0, b1, b2, b3, b4, idxv, bbuf, gsem, ssem, isem):
    wid = lax.axis_index("s") * 2 + lax.axis_index("c")
    t0 = wid * 2
    base = wid * _RPW

    pltpu.sync_copy(idxh.at[pl.ds(base, _RPW)], idxv)
    pltpu.async_copy(bh.at[idxv], bbuf, gsem).wait()
    pltpu.sync_copy(bbuf, bouth.at[pl.ds(base, _RPW)])

    bufs = (b0, b1, b2, b3, b4)
    ilead = 4
    glead = 2

    def start_i(u):
        ic, t = u >> 1, u & 1
        return pltpu.async_copy(nbth.at[t0 + t, pl.ds(ic * _C, _C)],
                                idxm.at[u % _RING], isem)

    def start_g(u):
        return pltpu.async_copy(xh.at[idxm.at[u % _RING]],
                                bufs[u % _RING], gsem)

    def start_s(u):
        ic, t = u >> 1, u & 1
        row0 = (t0 + t) * _MP + ic * _C
        return pltpu.async_copy(bufs[u % _RING],
                                xgh.at[pl.ds(row0, _C)], ssem)

    idxd = [None] * _NU
    scat = [None] * _NU
    gat = [None] * _NU
    for m in range(ilead):
        idxd[m] = start_i(m)
    for m in range(glead):
        idxd[m].wait()
        gat[m] = start_g(m)
    for u in range(_NU):
        ni = u + ilead
        ng = u + glead
        if ni < _NU:
            idxd[ni] = start_i(ni)
        if ng < _NU:
            if ng - _RING >= 0:
                scat[ng - _RING].wait()
            idxd[ng].wait()
            gat[ng] = start_g(ng)
        gat[u].wait()
        scat[u] = start_s(u)
    for u in range(max(_NU - _RING, 0), _NU):
        scat[u].wait()


def _gather(x, nbt, idxp, batch):
    mesh = plsc.VectorSubcoreMesh(core_axis_name="c", subcore_axis_name="s")
    f = pl.kernel(
        _gather_body,
        compiler_params=pltpu.CompilerParams(needs_layout_passes=False),
        out_type=(
            jax.ShapeDtypeStruct((64 * _MP, 128), jnp.float32),
            jax.ShapeDtypeStruct((_MP,), jnp.int32),
        ),
        mesh=mesh,
        scratch_types=[
            pltpu.VMEM((_RING, _C), jnp.int32),
            pltpu.VMEM((_C, 128), jnp.float32),
            pltpu.VMEM((_C, 128), jnp.float32),
            pltpu.VMEM((_C, 128), jnp.float32),
            pltpu.VMEM((_C, 128), jnp.float32),
            pltpu.VMEM((_C, 128), jnp.float32),
            pltpu.VMEM((_RPW,), jnp.int32),
            pltpu.VMEM((_RPW,), jnp.int32),
            pltpu.SemaphoreType.DMA,
            pltpu.SemaphoreType.DMA,
            pltpu.SemaphoreType.DMA,
        ],
    )
    return f(x, nbt, idxp, batch)


def _mlp_kernel(xg_ref, rx_ref, ry_ref, rz_ref, vm_ref, w1_ref, w2_ref,
                aux_ref, o_ref):
    w1 = w1_ref[...]
    w2 = w2_ref[...]
    aux = aux_ref[...]
    acc = jnp.full((128, 128), _NEG_INF, jnp.float32)
    for t in range(64):
        xt = xg_ref[t]
        h = jnp.dot(xt, w1, preferred_element_type=jnp.float32)
        h = h + rx_ref[:, t:t + 1] * aux[0:1, :]
        h = h + ry_ref[:, t:t + 1] * aux[1:2, :]
        h = h + rz_ref[:, t:t + 1] * aux[2:3, :]
        h = jnp.maximum(h + aux[3:4, :], 0.0)
        h2 = jnp.dot(h, w2, preferred_element_type=jnp.float32)
        h2 = jnp.maximum(h2 + aux[4:5, :], 0.0)
        acc = jnp.maximum(acc, h2 + vm_ref[:, t:t + 1])
    o_ref[...] = acc


def _mlp(xg, rx2, ry2, rz2, vm2, w1a, w2, aux):
    grid = (_MP // 128,)
    return pl.pallas_call(
        _mlp_kernel,
        grid=grid,
        in_specs=[
            pl.BlockSpec((64, 128, 128), lambda i: (0, i, 0)),
            pl.BlockSpec((128, 64), lambda i: (i, 0)),
            pl.BlockSpec((128, 64), lambda i: (i, 0)),
            pl.BlockSpec((128, 64), lambda i: (i, 0)),
            pl.BlockSpec((128, 64), lambda i: (i, 0)),
            pl.BlockSpec((128, 128), lambda i: (0, 0)),
            pl.BlockSpec((128, 128), lambda i: (0, 0)),
            pl.BlockSpec((8, 128), lambda i: (0, 0)),
        ],
        out_specs=pl.BlockSpec((128, 128), lambda i: (i, 0)),
        out_shape=jax.ShapeDtypeStruct((_MP, 128), jnp.float32),
    )(xg, rx2, ry2, rz2, vm2, w1a, w2, aux)


def kernel(x, pos, batch, W1, b1, W2, b2):
    idxf, qxf, qyf, qzf, pos_q = _fps(pos)

    big = jnp.float32(1e9)
    coords = jnp.concatenate(
        [pos, jnp.full((_NPAD - _N, 3), big, jnp.float32)], axis=0)
    px = coords[:, 0]
    py = coords[:, 1]
    pz = coords[:, 2]

    nbf, vmf, rxf, ryf, rzf = _ballquery(px, py, pz, qxf, qyf, qzf)

    nbt = _transpose_nb(nbf.reshape(_MP, 64))
    xg, bout = _gather(x, nbt, idxf, batch)

    aux = jnp.zeros((8, 128), jnp.float32)
    aux = aux.at[0:3, :].set(W1[128:131, :])
    aux = aux.at[3, :].set(b1)
    aux = aux.at[4, :].set(b2)

    out = _mlp(xg.reshape(64, _MP, 128),
               rxf.reshape(_MP, 64), ryf.reshape(_MP, 64),
               rzf.reshape(_MP, 64), vmf.reshape(_MP, 64),
               W1[:128, :], W2, aux)

    return (out[:_M], pos_q, bout[:_M])
